# Initial kernel scaffold; baseline (speedup 1.0000x reference)
import jax
import jax.numpy as jnp
from jax.experimental import pallas as pl


def kernel(x_chromophore, edge_index_chromophore, edge_attr_chromophore, x_solvent, edge_index_solvent, edge_attr_solvent, batch, W1_c, b1_c, g1_c, be1_c, W2_c, b2_c, g2_c, be2_c, W1_s, b1_s, g1_s, be1_s, W2_s, b2_s, g2_s, be2_s, fc1_W, fc1_b, gfc, befc, fc2_W, fc2_b):
    raise NotImplementedError("write your pallas kernel here")


# speedup vs baseline: 18.0558x; 18.0558x over previous
import functools

import jax
import jax.numpy as jnp
from jax import lax
from jax.experimental import pallas as pl
from jax.experimental.pallas import tpu as pltpu
from jax.experimental.pallas import tpu_sc as plsc

N = 10000
E = 320000
NG = 256
EPS = 1e-5

NC = 2
NS = 16
CH = 80
EPT = E // NS
NCHT = EPT // CH
NCH = E // CH
RPT = N // NS

NBLK = 10
BLK = N // NBLK


_MESH = plsc.VectorSubcoreMesh(core_axis_name="c", subcore_axis_name="s")
_SC_PARAMS = pltpu.CompilerParams(use_tc_tiling_on_sc=False)


@functools.partial(
    pl.kernel,
    out_type=jax.ShapeDtypeStruct((2 * N, 16), jnp.float32),
    mesh=_MESH,
    compiler_params=_SC_PARAMS,
    scratch_types=[
        pltpu.VMEM((NCHT, CH), jnp.int32),
        pltpu.VMEM((CH, 16), jnp.float32),
        pltpu.VMEM_SHARED((N, 16), jnp.float32),
    ],
)
def _sc_degree(dst_hbm, ones_hbm, zeros_hbm, out_hbm, dstb, onesb, acc):
    cid = lax.axis_index("c")
    sid = lax.axis_index("s")
    pltpu.sync_copy(dst_hbm.at[pl.ds(cid * NCH + sid * NCHT, NCHT)], dstb)
    pltpu.sync_copy(ones_hbm, onesb)
    r0 = sid * RPT
    pltpu.sync_copy(zeros_hbm.at[pl.ds(r0, RPT)], acc.at[pl.ds(r0, RPT)])
    plsc.subcore_barrier()

    def step(j, carry):
        pltpu.sync_copy(onesb, acc.at[dstb.at[j]], add=True)
        return carry

    lax.fori_loop(0, NCHT, step, 0)
    plsc.subcore_barrier()
    pltpu.sync_copy(acc.at[pl.ds(r0, RPT)], out_hbm.at[pl.ds(cid * N + r0, RPT)])


def _make_sc_conv(D):
    @functools.partial(
        pl.kernel,
        out_type=jax.ShapeDtypeStruct((2 * N, D), jnp.float32),
        mesh=_MESH,
        compiler_params=_SC_PARAMS,
        scratch_types=[
            pltpu.VMEM((EPT,), jnp.int32),
            pltpu.VMEM((NCHT, CH), jnp.int32),
            pltpu.VMEM((CH, D), jnp.float32),
            pltpu.VMEM_SHARED((N, D), jnp.float32),
        ],
    )
    def conv(h_hbm, src_hbm, dst_hbm, zeros_hbm, out_hbm, srcb, dstb, buf, acc):
        cid = lax.axis_index("c")
        sid = lax.axis_index("s")
        pltpu.sync_copy(src_hbm.at[pl.ds(cid * E + sid * EPT, EPT)], srcb)
        pltpu.sync_copy(dst_hbm.at[pl.ds(cid * NCH + sid * NCHT, NCHT)], dstb)
        r0 = sid * RPT
        pltpu.sync_copy(zeros_hbm.at[pl.ds(r0, RPT)], acc.at[pl.ds(r0, RPT)])
        plsc.subcore_barrier()

        def step(j, carry):
            pltpu.sync_copy(h_hbm.at[srcb.at[pl.ds(j * CH, CH)]], buf)
            pltpu.sync_copy(buf, acc.at[dstb.at[j]], add=True)
            return carry

        lax.fori_loop(0, NCHT, step, 0)
        plsc.subcore_barrier()
        pltpu.sync_copy(acc.at[pl.ds(r0, RPT)],
                        out_hbm.at[pl.ds(cid * N + r0, RPT)])

    return conv


_sc_conv128 = _make_sc_conv(128)
_sc_conv64 = _make_sc_conv(64)


def _row_spec(d):
    return pl.BlockSpec((BLK, d), lambda i: (i, 0))


def _full_spec(shape):
    nd = len(shape)
    return pl.BlockSpec(shape, lambda i, _n=nd: (0,) * _n)


def _dinv(deg_blk):
    return lax.rsqrt(deg_blk[:, :1] + 1.0)


def _tc1_body(xc, xs, w1c, w1s, dgc, dgs, hc_o, hs_o):
    hc_o[...] = jnp.dot(xc[...], w1c[...],
                        preferred_element_type=jnp.float32) * _dinv(dgc[...])
    hs_o[...] = jnp.dot(xs[...], w1s[...],
                        preferred_element_type=jnp.float32) * _dinv(dgs[...])


def _tc1(x_c, x_s, W1_c, W1_s, deg_c, deg_s):
    return pl.pallas_call(
        _tc1_body,
        grid=(NBLK,),
        in_specs=[_row_spec(128), _row_spec(128), _full_spec((128, 128)),
                  _full_spec((128, 128)), _row_spec(16), _row_spec(16)],
        out_specs=[_row_spec(128), _row_spec(128)],
        out_shape=[jax.ShapeDtypeStruct((N, 128), jnp.float32)] * 2,
    )(x_c, x_s, W1_c, W1_s, deg_c, deg_s)


def _make_tc_pre(D):
    def body(ac, as_, hc, hs, dgc, dgs, yc_o, ys_o, part_o):
        yc = _dinv(dgc[...]) * (ac[...] + hc[...])
        ys = _dinv(dgs[...]) * (as_[...] + hs[...])
        yc_o[...] = yc
        ys_o[...] = ys
        part_o[...] = jnp.stack([
            jnp.stack([jnp.sum(yc, 0), jnp.sum(yc * yc, 0)]),
            jnp.stack([jnp.sum(ys, 0), jnp.sum(ys * ys, 0)]),
        ])[None]

    def run(acc_c, acc_s, h_c, h_s, deg_c, deg_s):
        return pl.pallas_call(
            body,
            grid=(NBLK,),
            in_specs=[_row_spec(D)] * 4 + [_row_spec(16)] * 2,
            out_specs=[_row_spec(D), _row_spec(D),
                       pl.BlockSpec((1, 2, 2, D), lambda i: (i, 0, 0, 0))],
            out_shape=[jax.ShapeDtypeStruct((N, D), jnp.float32),
                       jax.ShapeDtypeStruct((N, D), jnp.float32),
                       jax.ShapeDtypeStruct((NBLK, 2, 2, D), jnp.float32)],
        )(acc_c, acc_s, h_c, h_s, deg_c, deg_s)

    return run


_tc_pre128 = _make_tc_pre(128)
_tc_pre64 = _make_tc_pre(64)


def _bn_apply(y, sums, g, be):
    mean = sums[0] / N
    var = sums[1] / N - mean * mean
    return (y - mean[None, :]) * lax.rsqrt(var[None, :] + EPS) * g[None, :] \
        + be[None, :]


def _tc2b_body(yc, ys, part, g1c, be1c, g1s, be1s, w2c, w2s, dgc, dgs,
               hc_o, hs_o):
    sums = jnp.sum(part[...], axis=0)
    xc = jnp.maximum(_bn_apply(yc[...], sums[0], g1c[...], be1c[...]), 0.0)
    xs = jnp.maximum(_bn_apply(ys[...], sums[1], g1s[...], be1s[...]), 0.0)
    hc_o[...] = jnp.dot(xc, w2c[...],
                        preferred_element_type=jnp.float32) * _dinv(dgc[...])
    hs_o[...] = jnp.dot(xs, w2s[...],
                        preferred_element_type=jnp.float32) * _dinv(dgs[...])


def _tc2b(y_c, y_s, part, g1c, be1c, g1s, be1s, W2_c, W2_s, deg_c, deg_s):
    return pl.pallas_call(
        _tc2b_body,
        grid=(NBLK,),
        in_specs=[_row_spec(128), _row_spec(128),
                  _full_spec((NBLK, 2, 2, 128)),
                  _full_spec((128,)), _full_spec((128,)),
                  _full_spec((128,)), _full_spec((128,)),
                  _full_spec((128, 64)), _full_spec((128, 64)),
                  _row_spec(16), _row_spec(16)],
        out_specs=[_row_spec(64), _row_spec(64)],
        out_shape=[jax.ShapeDtypeStruct((N, 64), jnp.float32)] * 2,
    )(y_c, y_s, part, g1c, be1c, g1s, be1s, W2_c, W2_s, deg_c, deg_s)


def _tc3b_body(yc, ys, part, g2c, be2c, g2s, be2s, batch, pool_o, cnt_o):
    sums = jnp.sum(part[...], axis=0)
    zc = jnp.maximum(_bn_apply(yc[...], sums[0], g2c[...], be2c[...]), 0.0)
    zs = jnp.maximum(_bn_apply(ys[...], sums[1], g2s[...], be2s[...]), 0.0)
    gids = lax.broadcasted_iota(jnp.int32, (NG, BLK), 0)
    onehot = (batch[...][0] == gids).astype(jnp.float32)
    pool_o[...] = jnp.stack([
        jnp.dot(onehot, zc, preferred_element_type=jnp.float32),
        jnp.dot(onehot, zs, preferred_element_type=jnp.float32),
    ])[None]
    cnt_o[...] = jnp.broadcast_to(
        jnp.sum(onehot, axis=1, keepdims=True), (NG, 16))[None]


def _tc3b(y2_c, y2_s, part, g2c, be2c, g2s, be2s, batch2d):
    return pl.pallas_call(
        _tc3b_body,
        grid=(NBLK,),
        in_specs=[_row_spec(64), _row_spec(64),
                  _full_spec((NBLK, 2, 2, 64)),
                  _full_spec((64,)), _full_spec((64,)),
                  _full_spec((64,)), _full_spec((64,)),
                  pl.BlockSpec((1, 1, BLK), lambda i: (i, 0, 0))],
        out_specs=[pl.BlockSpec((1, 2, NG, 64), lambda i: (i, 0, 0, 0)),
                   pl.BlockSpec((1, NG, 16), lambda i: (i, 0, 0))],
        out_shape=[jax.ShapeDtypeStruct((NBLK, 2, NG, 64), jnp.float32),
                   jax.ShapeDtypeStruct((NBLK, NG, 16), jnp.float32)],
    )(y2_c, y2_s, part, g2c, be2c, g2s, be2s, batch2d)


def _tc4_body(pool, cnt, fc1w, gfc, befc, fc2w, fc2b, out_o):
    psum = jnp.sum(pool[...], axis=0)
    c = jnp.maximum(jnp.sum(cnt[...], axis=0)[:, :1], 1.0)
    x = jnp.concatenate([psum[0] / c, psum[1] / c], axis=1)
    h = jnp.dot(x, fc1w[...], preferred_element_type=jnp.float32)
    m = jnp.mean(h, axis=0)
    v = jnp.mean((h - m[None, :]) ** 2, axis=0)
    h = (h - m[None, :]) * lax.rsqrt(v[None, :] + EPS) * gfc[...][None, :] \
        + befc[...][None, :]
    h = jnp.maximum(h, 0.0)
    out_o[...] = jnp.dot(h, fc2w[...],
                         preferred_element_type=jnp.float32) + fc2b[...][None, :]


def _tc4(pool, cnt, fc1_W, gfc, befc, fc2_Wp, fc2_bp):
    return pl.pallas_call(
        _tc4_body,
        grid=(1,),
        in_specs=[_full_spec((NBLK, 2, NG, 64)), _full_spec((NBLK, NG, 16)),
                  _full_spec((128, 64)), _full_spec((64,)), _full_spec((64,)),
                  _full_spec((64, 128)), _full_spec((128,))],
        out_specs=_full_spec((NG, 128)),
        out_shape=jax.ShapeDtypeStruct((NG, 128), jnp.float32),
    )(pool, cnt, fc1_W, gfc, befc, fc2_Wp, fc2_bp)


def kernel(x_chromophore, edge_index_chromophore, edge_attr_chromophore,
           x_solvent, edge_index_solvent, edge_attr_solvent, batch,
           W1_c, b1_c, g1_c, be1_c, W2_c, b2_c, g2_c, be2_c,
           W1_s, b1_s, g1_s, be1_s, W2_s, b2_s, g2_s, be2_s,
           fc1_W, fc1_b, gfc, befc, fc2_W, fc2_b):
    del edge_attr_chromophore, edge_attr_solvent
    del b1_c, b2_c, b1_s, b2_s, fc1_b

    src = jnp.concatenate([edge_index_chromophore[0],
                           edge_index_solvent[0] + N])
    dst2 = jnp.concatenate([edge_index_chromophore[1].reshape(NCH, CH),
                            edge_index_solvent[1].reshape(NCH, CH)])
    ones16 = jnp.ones((CH, 16), jnp.float32)
    z16 = jnp.zeros((N, 16), jnp.float32)
    z128 = jnp.zeros((N, 128), jnp.float32)
    z64 = jnp.zeros((N, 64), jnp.float32)

    degp = _sc_degree(dst2, ones16, z16)
    deg_c, deg_s = degp[:N], degp[N:]

    h1c, h1s = _tc1(x_chromophore, x_solvent, W1_c, W1_s, deg_c, deg_s)
    acc1 = _sc_conv128(jnp.concatenate([h1c, h1s]), src, dst2, z128)
    yc, ys, p1 = _tc_pre128(acc1[:N], acc1[N:], h1c, h1s, deg_c, deg_s)
    h2c, h2s = _tc2b(yc, ys, p1, g1_c, be1_c, g1_s, be1_s, W2_c, W2_s,
                     deg_c, deg_s)

    acc2 = _sc_conv64(jnp.concatenate([h2c, h2s]), src, dst2, z64)
    y2c, y2s, p2 = _tc_pre64(acc2[:N], acc2[N:], h2c, h2s, deg_c, deg_s)
    pool, cnt = _tc3b(y2c, y2s, p2, g2_c, be2_c, g2_s, be2_s,
                      batch.reshape(NBLK, 1, BLK))

    fc2_Wp = jnp.pad(fc2_W, ((0, 0), (0, 127)))
    fc2_bp = jnp.pad(fc2_b, (0, 127))
    out = _tc4(pool, cnt, fc1_W, gfc, befc, fc2_Wp, fc2_bp)
    return out[:, :1]


# speedup vs baseline: 26.9461x; 1.4924x over previous
import functools

import jax
import jax.numpy as jnp
from jax import lax
from jax.experimental import pallas as pl
from jax.experimental.pallas import tpu as pltpu
from jax.experimental.pallas import tpu_sc as plsc

N = 10000
E = 320000
NG = 256
EPS = 1e-5

NC = 2
NS = 16
CH = 80
EPT = E // NS
NCHT = EPT // CH
NCH = E // CH
RPT = N // NS

NBLK = 10
BLK = N // NBLK


_MESH = plsc.VectorSubcoreMesh(core_axis_name="c", subcore_axis_name="s")
_SC_PARAMS = pltpu.CompilerParams(use_tc_tiling_on_sc=False)


@functools.partial(
    pl.kernel,
    out_type=jax.ShapeDtypeStruct((2 * N, 16), jnp.float32),
    mesh=_MESH,
    compiler_params=_SC_PARAMS,
    scratch_types=[
        pltpu.VMEM((NCHT, CH), jnp.int32),
        pltpu.VMEM((CH, 16), jnp.float32),
        pltpu.VMEM_SHARED((N, 16), jnp.float32),
    ],
)
def _sc_degree(dst_hbm, ones_hbm, zeros_hbm, out_hbm, dstb, onesb, acc):
    cid = lax.axis_index("c")
    sid = lax.axis_index("s")
    pltpu.sync_copy(dst_hbm.at[pl.ds(cid * NCH + sid * NCHT, NCHT)], dstb)
    pltpu.sync_copy(ones_hbm, onesb)
    r0 = sid * RPT
    pltpu.sync_copy(zeros_hbm.at[pl.ds(r0, RPT)], acc.at[pl.ds(r0, RPT)])
    plsc.subcore_barrier()

    def step(j, carry):
        pltpu.sync_copy(onesb, acc.at[dstb.at[j]], add=True)
        return carry

    lax.fori_loop(0, NCHT, step, 0)
    plsc.subcore_barrier()
    pltpu.sync_copy(acc.at[pl.ds(r0, RPT)], out_hbm.at[pl.ds(cid * N + r0, RPT)])


def _make_sc_conv(D):
    @functools.partial(
        pl.kernel,
        out_type=jax.ShapeDtypeStruct((2 * N, D), jnp.float32),
        mesh=_MESH,
        compiler_params=_SC_PARAMS,
        scratch_types=[
            pltpu.VMEM((EPT // 2,), jnp.int32),
            pltpu.VMEM((NCHT // 2, CH), jnp.int32),
            pltpu.VMEM((CH, D), jnp.float32),
            pltpu.VMEM((CH, D), jnp.float32),
            pltpu.SemaphoreType.DMA,
            pltpu.SemaphoreType.DMA,
            pltpu.VMEM_SHARED((N, D), jnp.float32),
        ],
    )
    def conv(h_hbm, src_hbm, dst_hbm, zeros_hbm, out_hbm, srcb, dstb,
             buf_a, buf_b, sem_a, sem_b, acc):
        cid = lax.axis_index("c")
        sid = lax.axis_index("s")
        r0 = sid * RPT
        pltpu.sync_copy(zeros_hbm.at[pl.ds(r0, RPT)], acc.at[pl.ds(r0, RPT)])
        plsc.subcore_barrier()

        def gather_start(j, buf, sem):
            pltpu.async_copy(h_hbm.at[srcb.at[pl.ds(j * CH, CH)]], buf, sem)

        def gather_wait(j, buf, sem):
            pltpu.make_async_copy(h_hbm.at[srcb.at[pl.ds(j * CH, CH)]],
                                  buf, sem).wait()

        nh = NCHT // 2
        npair = nh // 2
        for h in range(2):
            pltpu.sync_copy(
                src_hbm.at[pl.ds(cid * E + sid * EPT + h * (EPT // 2),
                                 EPT // 2)], srcb)
            pltpu.sync_copy(
                dst_hbm.at[pl.ds(cid * NCH + sid * NCHT + h * nh, nh)], dstb)
            gather_start(0, buf_a, sem_a)

            def step(p, carry):
                j = 2 * p
                gather_start(j + 1, buf_b, sem_b)
                gather_wait(j, buf_a, sem_a)
                pltpu.sync_copy(buf_a, acc.at[dstb.at[j]], add=True)

                @pl.when(p < npair - 1)
                def _():
                    gather_start(j + 2, buf_a, sem_a)

                gather_wait(j + 1, buf_b, sem_b)
                pltpu.sync_copy(buf_b, acc.at[dstb.at[j + 1]], add=True)
                return carry

            lax.fori_loop(0, npair, step, 0)
            gather_start(nh - 1, buf_a, sem_a)
            gather_wait(nh - 1, buf_a, sem_a)
            pltpu.sync_copy(buf_a, acc.at[dstb.at[nh - 1]], add=True)
        plsc.subcore_barrier()
        pltpu.sync_copy(acc.at[pl.ds(r0, RPT)],
                        out_hbm.at[pl.ds(cid * N + r0, RPT)])

    return conv


_sc_conv128 = _make_sc_conv(128)
_sc_conv64 = _make_sc_conv(64)


def _row_spec(d):
    return pl.BlockSpec((BLK, d), lambda i: (i, 0))


def _full_spec(shape):
    nd = len(shape)
    return pl.BlockSpec(shape, lambda i, _n=nd: (0,) * _n)


def _dinv(deg_blk):
    return lax.rsqrt(deg_blk[:, :1] + 1.0)


def _tc1_body(xc, xs, w1c, w1s, dgc, dgs, hc_o, hs_o):
    hc_o[...] = jnp.dot(xc[...], w1c[...],
                        preferred_element_type=jnp.float32) * _dinv(dgc[...])
    hs_o[...] = jnp.dot(xs[...], w1s[...],
                        preferred_element_type=jnp.float32) * _dinv(dgs[...])


def _tc1(x_c, x_s, W1_c, W1_s, deg_c, deg_s):
    return pl.pallas_call(
        _tc1_body,
        grid=(NBLK,),
        in_specs=[_row_spec(128), _row_spec(128), _full_spec((128, 128)),
                  _full_spec((128, 128)), _row_spec(16), _row_spec(16)],
        out_specs=[_row_spec(128), _row_spec(128)],
        out_shape=[jax.ShapeDtypeStruct((N, 128), jnp.float32)] * 2,
    )(x_c, x_s, W1_c, W1_s, deg_c, deg_s)


def _make_tc_pre(D):
    def body(ac, as_, hc, hs, dgc, dgs, yc_o, ys_o, part_o):
        yc = _dinv(dgc[...]) * (ac[...] + hc[...])
        ys = _dinv(dgs[...]) * (as_[...] + hs[...])
        yc_o[...] = yc
        ys_o[...] = ys
        part_o[...] = jnp.stack([
            jnp.stack([jnp.sum(yc, 0), jnp.sum(yc * yc, 0)]),
            jnp.stack([jnp.sum(ys, 0), jnp.sum(ys * ys, 0)]),
        ])[None]

    def run(acc_c, acc_s, h_c, h_s, deg_c, deg_s):
        return pl.pallas_call(
            body,
            grid=(NBLK,),
            in_specs=[_row_spec(D)] * 4 + [_row_spec(16)] * 2,
            out_specs=[_row_spec(D), _row_spec(D),
                       pl.BlockSpec((1, 2, 2, D), lambda i: (i, 0, 0, 0))],
            out_shape=[jax.ShapeDtypeStruct((N, D), jnp.float32),
                       jax.ShapeDtypeStruct((N, D), jnp.float32),
                       jax.ShapeDtypeStruct((NBLK, 2, 2, D), jnp.float32)],
        )(acc_c, acc_s, h_c, h_s, deg_c, deg_s)

    return run


_tc_pre128 = _make_tc_pre(128)
_tc_pre64 = _make_tc_pre(64)


def _bn_apply(y, sums, g, be):
    mean = sums[0] / N
    var = sums[1] / N - mean * mean
    return (y - mean[None, :]) * lax.rsqrt(var[None, :] + EPS) * g[None, :] \
        + be[None, :]


def _tc2b_body(yc, ys, part, g1c, be1c, g1s, be1s, w2c, w2s, dgc, dgs,
               hc_o, hs_o):
    sums = jnp.sum(part[...], axis=0)
    xc = jnp.maximum(_bn_apply(yc[...], sums[0], g1c[...], be1c[...]), 0.0)
    xs = jnp.maximum(_bn_apply(ys[...], sums[1], g1s[...], be1s[...]), 0.0)
    hc_o[...] = jnp.dot(xc, w2c[...],
                        preferred_element_type=jnp.float32) * _dinv(dgc[...])
    hs_o[...] = jnp.dot(xs, w2s[...],
                        preferred_element_type=jnp.float32) * _dinv(dgs[...])


def _tc2b(y_c, y_s, part, g1c, be1c, g1s, be1s, W2_c, W2_s, deg_c, deg_s):
    return pl.pallas_call(
        _tc2b_body,
        grid=(NBLK,),
        in_specs=[_row_spec(128), _row_spec(128),
                  _full_spec((NBLK, 2, 2, 128)),
                  _full_spec((128,)), _full_spec((128,)),
                  _full_spec((128,)), _full_spec((128,)),
                  _full_spec((128, 64)), _full_spec((128, 64)),
                  _row_spec(16), _row_spec(16)],
        out_specs=[_row_spec(64), _row_spec(64)],
        out_shape=[jax.ShapeDtypeStruct((N, 64), jnp.float32)] * 2,
    )(y_c, y_s, part, g1c, be1c, g1s, be1s, W2_c, W2_s, deg_c, deg_s)


def _tc3b_body(yc, ys, part, g2c, be2c, g2s, be2s, batch, pool_o, cnt_o):
    sums = jnp.sum(part[...], axis=0)
    zc = jnp.maximum(_bn_apply(yc[...], sums[0], g2c[...], be2c[...]), 0.0)
    zs = jnp.maximum(_bn_apply(ys[...], sums[1], g2s[...], be2s[...]), 0.0)
    gids = lax.broadcasted_iota(jnp.int32, (NG, BLK), 0)
    onehot = (batch[...][0] == gids).astype(jnp.float32)
    pool_o[...] = jnp.stack([
        jnp.dot(onehot, zc, preferred_element_type=jnp.float32),
        jnp.dot(onehot, zs, preferred_element_type=jnp.float32),
    ])[None]
    cnt_o[...] = jnp.broadcast_to(
        jnp.sum(onehot, axis=1, keepdims=True), (NG, 16))[None]


def _tc3b(y2_c, y2_s, part, g2c, be2c, g2s, be2s, batch2d):
    return pl.pallas_call(
        _tc3b_body,
        grid=(NBLK,),
        in_specs=[_row_spec(64), _row_spec(64),
                  _full_spec((NBLK, 2, 2, 64)),
                  _full_spec((64,)), _full_spec((64,)),
                  _full_spec((64,)), _full_spec((64,)),
                  pl.BlockSpec((1, 1, BLK), lambda i: (i, 0, 0))],
        out_specs=[pl.BlockSpec((1, 2, NG, 64), lambda i: (i, 0, 0, 0)),
                   pl.BlockSpec((1, NG, 16), lambda i: (i, 0, 0))],
        out_shape=[jax.ShapeDtypeStruct((NBLK, 2, NG, 64), jnp.float32),
                   jax.ShapeDtypeStruct((NBLK, NG, 16), jnp.float32)],
    )(y2_c, y2_s, part, g2c, be2c, g2s, be2s, batch2d)


def _tc4_body(pool, cnt, fc1w, gfc, befc, fc2w, fc2b, out_o):
    psum = jnp.sum(pool[...], axis=0)
    c = jnp.maximum(jnp.sum(cnt[...], axis=0)[:, :1], 1.0)
    x = jnp.concatenate([psum[0] / c, psum[1] / c], axis=1)
    h = jnp.dot(x, fc1w[...], preferred_element_type=jnp.float32)
    m = jnp.mean(h, axis=0)
    v = jnp.mean((h - m[None, :]) ** 2, axis=0)
    h = (h - m[None, :]) * lax.rsqrt(v[None, :] + EPS) * gfc[...][None, :] \
        + befc[...][None, :]
    h = jnp.maximum(h, 0.0)
    out_o[...] = jnp.dot(h, fc2w[...],
                         preferred_element_type=jnp.float32) + fc2b[...][None, :]


def _tc4(pool, cnt, fc1_W, gfc, befc, fc2_Wp, fc2_bp):
    return pl.pallas_call(
        _tc4_body,
        grid=(1,),
        in_specs=[_full_spec((NBLK, 2, NG, 64)), _full_spec((NBLK, NG, 16)),
                  _full_spec((128, 64)), _full_spec((64,)), _full_spec((64,)),
                  _full_spec((64, 128)), _full_spec((128,))],
        out_specs=_full_spec((NG, 128)),
        out_shape=jax.ShapeDtypeStruct((NG, 128), jnp.float32),
    )(pool, cnt, fc1_W, gfc, befc, fc2_Wp, fc2_bp)


def kernel(x_chromophore, edge_index_chromophore, edge_attr_chromophore,
           x_solvent, edge_index_solvent, edge_attr_solvent, batch,
           W1_c, b1_c, g1_c, be1_c, W2_c, b2_c, g2_c, be2_c,
           W1_s, b1_s, g1_s, be1_s, W2_s, b2_s, g2_s, be2_s,
           fc1_W, fc1_b, gfc, befc, fc2_W, fc2_b):
    del edge_attr_chromophore, edge_attr_solvent
    del b1_c, b2_c, b1_s, b2_s, fc1_b

    src = jnp.concatenate([edge_index_chromophore[0],
                           edge_index_solvent[0] + N])
    dst2 = jnp.concatenate([edge_index_chromophore[1].reshape(NCH, CH),
                            edge_index_solvent[1].reshape(NCH, CH)])
    ones16 = jnp.ones((CH, 16), jnp.float32)
    z16 = jnp.zeros((N, 16), jnp.float32)
    z128 = jnp.zeros((N, 128), jnp.float32)
    z64 = jnp.zeros((N, 64), jnp.float32)

    degp = _sc_degree(dst2, ones16, z16)
    deg_c, deg_s = degp[:N], degp[N:]

    h1c, h1s = _tc1(x_chromophore, x_solvent, W1_c, W1_s, deg_c, deg_s)
    acc1 = _sc_conv128(jnp.concatenate([h1c, h1s]), src, dst2, z128)
    yc, ys, p1 = _tc_pre128(acc1[:N], acc1[N:], h1c, h1s, deg_c, deg_s)
    h2c, h2s = _tc2b(yc, ys, p1, g1_c, be1_c, g1_s, be1_s, W2_c, W2_s,
                     deg_c, deg_s)

    acc2 = _sc_conv64(jnp.concatenate([h2c, h2s]), src, dst2, z64)
    y2c, y2s, p2 = _tc_pre64(acc2[:N], acc2[N:], h2c, h2s, deg_c, deg_s)
    pool, cnt = _tc3b(y2c, y2s, p2, g2_c, be2_c, g2_s, be2_s,
                      batch.reshape(NBLK, 1, BLK))

    fc2_Wp = jnp.pad(fc2_W, ((0, 0), (0, 127)))
    fc2_bp = jnp.pad(fc2_b, (0, 127))
    out = _tc4(pool, cnt, fc1_W, gfc, befc, fc2_Wp, fc2_bp)
    return out[:, :1]


# speedup vs baseline: 27.3416x; 1.0147x over previous
import functools

import jax
import jax.numpy as jnp
from jax import lax
from jax.experimental import pallas as pl
from jax.experimental.pallas import tpu as pltpu
from jax.experimental.pallas import tpu_sc as plsc

N = 10000
E = 320000
NG = 256
EPS = 1e-5

NC = 2
NS = 16
CH = 80
EPT = E // NS
NCHT = EPT // CH
NCH = E // CH
RPT = N // NS

NBLK = 10
BLK = N // NBLK


_MESH = plsc.VectorSubcoreMesh(core_axis_name="c", subcore_axis_name="s")
_SC_PARAMS = pltpu.CompilerParams(use_tc_tiling_on_sc=False)


@functools.partial(
    pl.kernel,
    out_type=jax.ShapeDtypeStruct((2 * N, 16), jnp.float32),
    mesh=_MESH,
    compiler_params=_SC_PARAMS,
    scratch_types=[
        pltpu.VMEM((NCHT, CH), jnp.int32),
        pltpu.VMEM((CH, 16), jnp.float32),
        pltpu.VMEM_SHARED((N, 16), jnp.float32),
    ],
)
def _sc_degree(dst_hbm, ones_hbm, zeros_hbm, out_hbm, dstb, onesb, acc):
    cid = lax.axis_index("c")
    sid = lax.axis_index("s")
    pltpu.sync_copy(dst_hbm.at[pl.ds(cid * NCH + sid * NCHT, NCHT)], dstb)
    pltpu.sync_copy(ones_hbm, onesb)
    r0 = sid * RPT
    pltpu.sync_copy(zeros_hbm.at[pl.ds(r0, RPT)], acc.at[pl.ds(r0, RPT)])
    plsc.subcore_barrier()

    def step(j, carry):
        pltpu.sync_copy(onesb, acc.at[dstb.at[j]], add=True)
        return carry

    lax.fori_loop(0, NCHT, step, 0)
    plsc.subcore_barrier()
    pltpu.sync_copy(acc.at[pl.ds(r0, RPT)], out_hbm.at[pl.ds(cid * N + r0, RPT)])


def _make_sc_conv(D):
    @functools.partial(
        pl.kernel,
        out_type=jax.ShapeDtypeStruct((2 * N, D), jnp.float32),
        mesh=_MESH,
        compiler_params=_SC_PARAMS,
        scratch_types=[
            pltpu.VMEM((EPT // 2,), jnp.int32),
            pltpu.VMEM((NCHT // 2, CH), jnp.int32),
            pltpu.VMEM((CH, D), jnp.float32),
            pltpu.VMEM((CH, D), jnp.float32),
            pltpu.SemaphoreType.DMA,
            pltpu.SemaphoreType.DMA,
            pltpu.VMEM_SHARED((N, D), jnp.float32),
        ],
    )
    def conv(h_hbm, src_hbm, dst_hbm, zeros_hbm, out_hbm, srcb, dstb,
             buf_a, buf_b, sem_a, sem_b, acc):
        cid = lax.axis_index("c")
        sid = lax.axis_index("s")
        r0 = sid * RPT
        pltpu.sync_copy(zeros_hbm.at[pl.ds(r0, RPT)], acc.at[pl.ds(r0, RPT)])
        plsc.subcore_barrier()

        def gather_start(j, buf, sem):
            pltpu.async_copy(h_hbm.at[srcb.at[pl.ds(j * CH, CH)]], buf, sem)

        def gather_wait(j, buf, sem):
            pltpu.make_async_copy(h_hbm.at[srcb.at[pl.ds(j * CH, CH)]],
                                  buf, sem).wait()

        nh = NCHT // 2
        npair = nh // 2
        for h in range(2):
            pltpu.sync_copy(
                src_hbm.at[pl.ds(cid * E + sid * EPT + h * (EPT // 2),
                                 EPT // 2)], srcb)
            pltpu.sync_copy(
                dst_hbm.at[pl.ds(cid * NCH + sid * NCHT + h * nh, nh)], dstb)
            gather_start(0, buf_a, sem_a)

            def step(p, carry):
                j = 2 * p
                gather_start(j + 1, buf_b, sem_b)
                gather_wait(j, buf_a, sem_a)
                pltpu.sync_copy(buf_a, acc.at[dstb.at[j]], add=True)

                @pl.when(p < npair - 1)
                def _():
                    gather_start(j + 2, buf_a, sem_a)

                gather_wait(j + 1, buf_b, sem_b)
                pltpu.sync_copy(buf_b, acc.at[dstb.at[j + 1]], add=True)
                return carry

            lax.fori_loop(0, npair, step, 0)
            gather_start(nh - 1, buf_a, sem_a)
            gather_wait(nh - 1, buf_a, sem_a)
            pltpu.sync_copy(buf_a, acc.at[dstb.at[nh - 1]], add=True)
        plsc.subcore_barrier()
        pltpu.sync_copy(acc.at[pl.ds(r0, RPT)],
                        out_hbm.at[pl.ds(cid * N + r0, RPT)])

    return conv


_sc_conv128 = _make_sc_conv(128)
_sc_conv64 = _make_sc_conv(64)


NSTEP = 2 * NBLK


def _row_spec(d):
    return pl.BlockSpec((BLK, d), lambda i: (i, 0))


def _br_spec(shape):
    nd = len(shape)
    return pl.BlockSpec((1,) + shape, lambda i, _n=nd: (i // NBLK,) + (0,) * _n)


def _full_spec(shape):
    nd = len(shape)
    return pl.BlockSpec(shape, lambda i, _n=nd: (0,) * _n)


def _dinv(deg_blk):
    return lax.rsqrt(deg_blk[:, :1] + 1.0)


def _branch_sums(part_full):
    b = pl.program_id(0) // NBLK
    sel = (lax.broadcasted_iota(jnp.int32, (NSTEP, 1, 1), 0) // NBLK == b)
    return jnp.sum(jnp.where(sel, part_full, 0.0), axis=0)


def _sel_row(v2d):
    b = pl.program_id(0) // NBLK
    sel = (lax.broadcasted_iota(jnp.int32, (2, 1), 0) == b)
    return jnp.sum(jnp.where(sel, v2d, 0.0), axis=0)


def _tc1_body(x, w1, dg, h_o):
    h_o[...] = jnp.dot(x[...], w1[...][0],
                       preferred_element_type=jnp.float32) * _dinv(dg[...])


def _tc1(xall, W1, degp):
    return pl.pallas_call(
        _tc1_body,
        grid=(NSTEP,),
        in_specs=[_row_spec(128), _br_spec((128, 128)), _row_spec(16)],
        out_specs=_row_spec(128),
        out_shape=jax.ShapeDtypeStruct((2 * N, 128), jnp.float32),
    )(xall, W1, degp)


def _make_tc_pre(D):
    def body(ac, hc, dg, y_o, part_o):
        y = _dinv(dg[...]) * (ac[...] + hc[...])
        y_o[...] = y
        part_o[...] = jnp.stack([jnp.sum(y, 0), jnp.sum(y * y, 0)])[None]

    def run(acc, h, degp):
        return pl.pallas_call(
            body,
            grid=(NSTEP,),
            in_specs=[_row_spec(D), _row_spec(D), _row_spec(16)],
            out_specs=[_row_spec(D),
                       pl.BlockSpec((1, 2, D), lambda i: (i, 0, 0))],
            out_shape=[jax.ShapeDtypeStruct((2 * N, D), jnp.float32),
                       jax.ShapeDtypeStruct((NSTEP, 2, D), jnp.float32)],
        )(acc, h, degp)

    return run


_tc_pre128 = _make_tc_pre(128)
_tc_pre64 = _make_tc_pre(64)


def _bn_apply(y, sums, g, be):
    mean = sums[0] / N
    var = sums[1] / N - mean * mean
    return (y - mean[None, :]) * lax.rsqrt(var[None, :] + EPS) * g[None, :] \
        + be[None, :]


def _tc2b_body(y, part, g1, be1, w2, dg, h_o):
    sums = _branch_sums(part[...])
    x = jnp.maximum(_bn_apply(y[...], sums, _sel_row(g1[...]),
                              _sel_row(be1[...])), 0.0)
    h_o[...] = jnp.dot(x, w2[...][0],
                       preferred_element_type=jnp.float32) * _dinv(dg[...])


def _tc2b(y, part, g1, be1, W2, degp):
    return pl.pallas_call(
        _tc2b_body,
        grid=(NSTEP,),
        in_specs=[_row_spec(128), _full_spec((NSTEP, 2, 128)),
                  _full_spec((2, 128)), _full_spec((2, 128)),
                  _br_spec((128, 64)), _row_spec(16)],
        out_specs=_row_spec(64),
        out_shape=jax.ShapeDtypeStruct((2 * N, 64), jnp.float32),
    )(y, part, g1, be1, W2, degp)


def _tc3b_body(y, part, g2, be2, batch, pool_o, cnt_o):
    sums = _branch_sums(part[...])
    z = jnp.maximum(_bn_apply(y[...], sums, _sel_row(g2[...]),
                              _sel_row(be2[...])), 0.0)
    gids = lax.broadcasted_iota(jnp.int32, (NG, BLK), 0)
    onehot = (batch[...][0] == gids).astype(jnp.float32)
    pool_o[...] = jnp.dot(onehot, z, preferred_element_type=jnp.float32)[None]
    cnt_o[...] = jnp.broadcast_to(
        jnp.sum(onehot, axis=1, keepdims=True), (NG, 16))[None]


def _tc3b(y2, part, g2, be2, batch3d):
    return pl.pallas_call(
        _tc3b_body,
        grid=(NSTEP,),
        in_specs=[_row_spec(64), _full_spec((NSTEP, 2, 64)),
                  _full_spec((2, 64)), _full_spec((2, 64)),
                  pl.BlockSpec((1, 1, BLK), lambda i: (i % NBLK, 0, 0))],
        out_specs=[pl.BlockSpec((1, NG, 64), lambda i: (i, 0, 0)),
                   pl.BlockSpec((1, NG, 16), lambda i: (i % NBLK, 0, 0))],
        out_shape=[jax.ShapeDtypeStruct((NSTEP, NG, 64), jnp.float32),
                   jax.ShapeDtypeStruct((NBLK, NG, 16), jnp.float32)],
    )(y2, part, g2, be2, batch3d)


def _tc4_body(pool, cnt, fc1w, gfc, befc, fc2w, fc2b, out_o):
    pall = pool[...]
    c = jnp.maximum(jnp.sum(cnt[...], axis=0)[:, :1], 1.0)
    x = jnp.concatenate([jnp.sum(pall[:NBLK], 0) / c,
                         jnp.sum(pall[NBLK:], 0) / c], axis=1)
    h = jnp.dot(x, fc1w[...], preferred_element_type=jnp.float32)
    m = jnp.mean(h, axis=0)
    v = jnp.mean((h - m[None, :]) ** 2, axis=0)
    h = (h - m[None, :]) * lax.rsqrt(v[None, :] + EPS) * gfc[...][None, :] \
        + befc[...][None, :]
    h = jnp.maximum(h, 0.0)
    out_o[...] = jnp.dot(h, fc2w[...],
                         preferred_element_type=jnp.float32) + fc2b[...][None, :]


def _tc4(pool, cnt, fc1_W, gfc, befc, fc2_Wp, fc2_bp):
    return pl.pallas_call(
        _tc4_body,
        grid=(1,),
        in_specs=[_full_spec((NSTEP, NG, 64)), _full_spec((NBLK, NG, 16)),
                  _full_spec((128, 64)), _full_spec((64,)), _full_spec((64,)),
                  _full_spec((64, 128)), _full_spec((128,))],
        out_specs=_full_spec((NG, 128)),
        out_shape=jax.ShapeDtypeStruct((NG, 128), jnp.float32),
    )(pool, cnt, fc1_W, gfc, befc, fc2_Wp, fc2_bp)


def kernel(x_chromophore, edge_index_chromophore, edge_attr_chromophore,
           x_solvent, edge_index_solvent, edge_attr_solvent, batch,
           W1_c, b1_c, g1_c, be1_c, W2_c, b2_c, g2_c, be2_c,
           W1_s, b1_s, g1_s, be1_s, W2_s, b2_s, g2_s, be2_s,
           fc1_W, fc1_b, gfc, befc, fc2_W, fc2_b):
    del edge_attr_chromophore, edge_attr_solvent
    del b1_c, b2_c, b1_s, b2_s, fc1_b

    src = jnp.concatenate([edge_index_chromophore[0],
                           edge_index_solvent[0] + N])
    dst2 = jnp.concatenate([edge_index_chromophore[1].reshape(NCH, CH),
                            edge_index_solvent[1].reshape(NCH, CH)])
    ones16 = jnp.ones((CH, 16), jnp.float32)
    z16 = jnp.zeros((N, 16), jnp.float32)
    z128 = jnp.zeros((N, 128), jnp.float32)
    z64 = jnp.zeros((N, 64), jnp.float32)

    degp = _sc_degree(dst2, ones16, z16)

    xall = jnp.concatenate([x_chromophore, x_solvent])
    h1 = _tc1(xall, jnp.stack([W1_c, W1_s]), degp)
    acc1 = _sc_conv128(h1, src, dst2, z128)
    y1, p1 = _tc_pre128(acc1, h1, degp)
    h2 = _tc2b(y1, p1, jnp.stack([g1_c, g1_s]), jnp.stack([be1_c, be1_s]),
               jnp.stack([W2_c, W2_s]), degp)

    acc2 = _sc_conv64(h2, src, dst2, z64)
    y2, p2 = _tc_pre64(acc2, h2, degp)
    pool, cnt = _tc3b(y2, p2, jnp.stack([g2_c, g2_s]),
                      jnp.stack([be2_c, be2_s]), batch.reshape(NBLK, 1, BLK))

    fc2_Wp = jnp.pad(fc2_W, ((0, 0), (0, 127)))
    fc2_bp = jnp.pad(fc2_b, (0, 127))
    out = _tc4(pool, cnt, fc1_W, gfc, befc, fc2_Wp, fc2_bp)
    return out[:, :1]


# speedup vs baseline: 32.9315x; 1.2044x over previous
import functools

import jax
import jax.numpy as jnp
from jax import lax
from jax.experimental import pallas as pl
from jax.experimental.pallas import tpu as pltpu
from jax.experimental.pallas import tpu_sc as plsc

N = 10000
E = 320000
NG = 256
EPS = 1e-5

NC = 2
NS = 16
CH = 80
EPT = E // NS
NCHT = EPT // CH
NCH = E // CH
RPT = N // NS

NBLK = 10
BLK = N // NBLK


_MESH = plsc.VectorSubcoreMesh(core_axis_name="c", subcore_axis_name="s")
_SC_PARAMS = pltpu.CompilerParams(use_tc_tiling_on_sc=False)


@functools.partial(
    pl.kernel,
    out_type=jax.ShapeDtypeStruct((2 * N, 16), jnp.float32),
    mesh=_MESH,
    compiler_params=_SC_PARAMS,
    scratch_types=[
        pltpu.VMEM((NCHT, CH), jnp.int32),
        pltpu.VMEM((CH, 16), jnp.float32),
        pltpu.SemaphoreType.DMA,
        pltpu.VMEM_SHARED((N, 16), jnp.float32),
    ],
)
def _sc_degree(dst_hbm, ones_hbm, zeros_hbm, out_hbm, dstb, onesb, sem, acc):
    cid = lax.axis_index("c")
    sid = lax.axis_index("s")
    pltpu.sync_copy(dst_hbm.at[pl.ds(cid * NCH + sid * NCHT, NCHT)], dstb)
    pltpu.sync_copy(ones_hbm, onesb)
    r0 = sid * RPT
    pltpu.sync_copy(zeros_hbm.at[pl.ds(r0, RPT)], acc.at[pl.ds(r0, RPT)])
    plsc.subcore_barrier()

    def fire(j, carry):
        pltpu.async_copy(onesb, acc.at[dstb.at[j]], sem, add=True)
        return carry

    lax.fori_loop(0, NCHT, fire, 0)

    def drain(j, carry):
        pltpu.make_async_copy(onesb, acc.at[dstb.at[j]], sem).wait()
        return carry

    lax.fori_loop(0, NCHT, drain, 0)
    plsc.subcore_barrier()
    pltpu.sync_copy(acc.at[pl.ds(r0, RPT)], out_hbm.at[pl.ds(cid * N + r0, RPT)])


def _make_sc_conv(D):
    @functools.partial(
        pl.kernel,
        out_type=jax.ShapeDtypeStruct((2 * N, D), jnp.float32),
        mesh=_MESH,
        compiler_params=_SC_PARAMS,
        scratch_types=[
            pltpu.VMEM((EPT // 2,), jnp.int32),
            pltpu.VMEM((NCHT // 2, CH), jnp.int32),
            pltpu.VMEM((CH, D), jnp.float32),
            pltpu.VMEM((CH, D), jnp.float32),
            pltpu.VMEM((CH, D), jnp.float32),
            pltpu.SemaphoreType.DMA,
            pltpu.SemaphoreType.DMA,
            pltpu.SemaphoreType.DMA,
            pltpu.SemaphoreType.DMA,
            pltpu.SemaphoreType.DMA,
            pltpu.SemaphoreType.DMA,
            pltpu.VMEM_SHARED((N, D), jnp.float32),
        ],
    )
    def conv(h_hbm, src_hbm, dst_hbm, zeros_hbm, out_hbm, srcb, dstb,
             b0, b1, b2, g0, g1, g2, s0, s1, s2, acc):
        cid = lax.axis_index("c")
        sid = lax.axis_index("s")
        r0 = sid * RPT
        pltpu.sync_copy(zeros_hbm.at[pl.ds(r0, RPT)], acc.at[pl.ds(r0, RPT)])
        plsc.subcore_barrier()
        bufs = (b0, b1, b2)
        gsems = (g0, g1, g2)
        ssems = (s0, s1, s2)

        def g_start(j, k):
            pltpu.async_copy(h_hbm.at[srcb.at[pl.ds(j * CH, CH)]],
                             bufs[k], gsems[k])

        def g_wait(j, k):
            pltpu.make_async_copy(h_hbm.at[srcb.at[pl.ds(j * CH, CH)]],
                                  bufs[k], gsems[k]).wait()

        def s_start(j, k):
            pltpu.async_copy(bufs[k], acc.at[dstb.at[j]], ssems[k], add=True)

        def s_drain(j, k):
            pltpu.make_async_copy(bufs[k], acc.at[dstb.at[j]],
                                  ssems[k]).wait()

        nh = NCHT // 2
        nloop = (nh - 2) // 3
        for h in range(2):
            pltpu.sync_copy(
                src_hbm.at[pl.ds(cid * E + sid * EPT + h * (EPT // 2),
                                 EPT // 2)], srcb)
            pltpu.sync_copy(
                dst_hbm.at[pl.ds(cid * NCH + sid * NCHT + h * nh, nh)], dstb)
            for k in range(3):
                g_start(k, k)

            def step(t, carry):
                for k in range(3):
                    j = 3 * t + k
                    g_wait(j, k)
                    s_start(j, k)

                    @pl.when(j + 3 < nh)
                    def _(j=j, k=k):
                        s_drain(j, k)
                        g_start(j + 3, k)
                return carry

            lax.fori_loop(0, nloop, step, 0)
            jt = nh - 2
            g_wait(jt, 0)
            s_start(jt, 0)
            g_wait(jt + 1, 1)
            s_start(jt + 1, 1)
            s_drain(jt - 1, 2)
            s_drain(jt, 0)
            s_drain(jt + 1, 1)
        plsc.subcore_barrier()
        pltpu.sync_copy(acc.at[pl.ds(r0, RPT)],
                        out_hbm.at[pl.ds(cid * N + r0, RPT)])

    return conv


_sc_conv128 = _make_sc_conv(128)
_sc_conv64 = _make_sc_conv(64)


NSTEP = 2 * NBLK


def _row_spec(d):
    return pl.BlockSpec((BLK, d), lambda i: (i, 0))


def _br_spec(shape):
    nd = len(shape)
    return pl.BlockSpec((1,) + shape, lambda i, _n=nd: (i // NBLK,) + (0,) * _n)


def _full_spec(shape):
    nd = len(shape)
    return pl.BlockSpec(shape, lambda i, _n=nd: (0,) * _n)


def _dinv(deg_blk):
    return lax.rsqrt(deg_blk[:, :1] + 1.0)


def _branch_sums(part_full):
    b = pl.program_id(0) // NBLK
    sel = (lax.broadcasted_iota(jnp.int32, (NSTEP, 1, 1), 0) // NBLK == b)
    return jnp.sum(jnp.where(sel, part_full, 0.0), axis=0)


def _sel_row(v2d):
    b = pl.program_id(0) // NBLK
    sel = (lax.broadcasted_iota(jnp.int32, (2, 1), 0) == b)
    return jnp.sum(jnp.where(sel, v2d, 0.0), axis=0)


def _tc1_body(x, w1, dg, h_o):
    h_o[...] = jnp.dot(x[...], w1[...][0],
                       preferred_element_type=jnp.float32) * _dinv(dg[...])


def _tc1(xall, W1, degp):
    return pl.pallas_call(
        _tc1_body,
        grid=(NSTEP,),
        in_specs=[_row_spec(128), _br_spec((128, 128)), _row_spec(16)],
        out_specs=_row_spec(128),
        out_shape=jax.ShapeDtypeStruct((2 * N, 128), jnp.float32),
    )(xall, W1, degp)


def _make_tc_pre(D):
    def body(ac, hc, dg, y_o, part_o):
        y = _dinv(dg[...]) * (ac[...] + hc[...])
        y_o[...] = y
        part_o[...] = jnp.stack([jnp.sum(y, 0), jnp.sum(y * y, 0)])[None]

    def run(acc, h, degp):
        return pl.pallas_call(
            body,
            grid=(NSTEP,),
            in_specs=[_row_spec(D), _row_spec(D), _row_spec(16)],
            out_specs=[_row_spec(D),
                       pl.BlockSpec((1, 2, D), lambda i: (i, 0, 0))],
            out_shape=[jax.ShapeDtypeStruct((2 * N, D), jnp.float32),
                       jax.ShapeDtypeStruct((NSTEP, 2, D), jnp.float32)],
        )(acc, h, degp)

    return run


_tc_pre128 = _make_tc_pre(128)
_tc_pre64 = _make_tc_pre(64)


def _bn_apply(y, sums, g, be):
    mean = sums[0] / N
    var = sums[1] / N - mean * mean
    return (y - mean[None, :]) * lax.rsqrt(var[None, :] + EPS) * g[None, :] \
        + be[None, :]


def _tc2b_body(y, part, g1, be1, w2, dg, h_o):
    sums = _branch_sums(part[...])
    x = jnp.maximum(_bn_apply(y[...], sums, _sel_row(g1[...]),
                              _sel_row(be1[...])), 0.0)
    h_o[...] = jnp.dot(x, w2[...][0],
                       preferred_element_type=jnp.float32) * _dinv(dg[...])


def _tc2b(y, part, g1, be1, W2, degp):
    return pl.pallas_call(
        _tc2b_body,
        grid=(NSTEP,),
        in_specs=[_row_spec(128), _full_spec((NSTEP, 2, 128)),
                  _full_spec((2, 128)), _full_spec((2, 128)),
                  _br_spec((128, 64)), _row_spec(16)],
        out_specs=_row_spec(64),
        out_shape=jax.ShapeDtypeStruct((2 * N, 64), jnp.float32),
    )(y, part, g1, be1, W2, degp)


def _tc3b_body(y, part, g2, be2, batch, pool_o, cnt_o):
    sums = _branch_sums(part[...])
    z = jnp.maximum(_bn_apply(y[...], sums, _sel_row(g2[...]),
                              _sel_row(be2[...])), 0.0)
    gids = lax.broadcasted_iota(jnp.int32, (NG, BLK), 0)
    onehot = (batch[...][0] == gids).astype(jnp.float32)
    pool_o[...] = jnp.dot(onehot, z, preferred_element_type=jnp.float32)[None]
    cnt_o[...] = jnp.broadcast_to(
        jnp.sum(onehot, axis=1, keepdims=True), (NG, 16))[None]


def _tc3b(y2, part, g2, be2, batch3d):
    return pl.pallas_call(
        _tc3b_body,
        grid=(NSTEP,),
        in_specs=[_row_spec(64), _full_spec((NSTEP, 2, 64)),
                  _full_spec((2, 64)), _full_spec((2, 64)),
                  pl.BlockSpec((1, 1, BLK), lambda i: (i % NBLK, 0, 0))],
        out_specs=[pl.BlockSpec((1, NG, 64), lambda i: (i, 0, 0)),
                   pl.BlockSpec((1, NG, 16), lambda i: (i % NBLK, 0, 0))],
        out_shape=[jax.ShapeDtypeStruct((NSTEP, NG, 64), jnp.float32),
                   jax.ShapeDtypeStruct((NBLK, NG, 16), jnp.float32)],
    )(y2, part, g2, be2, batch3d)


def _tc4_body(pool, cnt, fc1w, gfc, befc, fc2w, fc2b, out_o):
    pall = pool[...]
    c = jnp.maximum(jnp.sum(cnt[...], axis=0)[:, :1], 1.0)
    x = jnp.concatenate([jnp.sum(pall[:NBLK], 0) / c,
                         jnp.sum(pall[NBLK:], 0) / c], axis=1)
    h = jnp.dot(x, fc1w[...], preferred_element_type=jnp.float32)
    m = jnp.mean(h, axis=0)
    v = jnp.mean((h - m[None, :]) ** 2, axis=0)
    h = (h - m[None, :]) * lax.rsqrt(v[None, :] + EPS) * gfc[...][None, :] \
        + befc[...][None, :]
    h = jnp.maximum(h, 0.0)
    out_o[...] = jnp.dot(h, fc2w[...],
                         preferred_element_type=jnp.float32) + fc2b[...][None, :]


def _tc4(pool, cnt, fc1_W, gfc, befc, fc2_Wp, fc2_bp):
    return pl.pallas_call(
        _tc4_body,
        grid=(1,),
        in_specs=[_full_spec((NSTEP, NG, 64)), _full_spec((NBLK, NG, 16)),
                  _full_spec((128, 64)), _full_spec((64,)), _full_spec((64,)),
                  _full_spec((64, 128)), _full_spec((128,))],
        out_specs=_full_spec((NG, 128)),
        out_shape=jax.ShapeDtypeStruct((NG, 128), jnp.float32),
    )(pool, cnt, fc1_W, gfc, befc, fc2_Wp, fc2_bp)


def kernel(x_chromophore, edge_index_chromophore, edge_attr_chromophore,
           x_solvent, edge_index_solvent, edge_attr_solvent, batch,
           W1_c, b1_c, g1_c, be1_c, W2_c, b2_c, g2_c, be2_c,
           W1_s, b1_s, g1_s, be1_s, W2_s, b2_s, g2_s, be2_s,
           fc1_W, fc1_b, gfc, befc, fc2_W, fc2_b):
    del edge_attr_chromophore, edge_attr_solvent
    del b1_c, b2_c, b1_s, b2_s, fc1_b

    src = jnp.concatenate([edge_index_chromophore[0],
                           edge_index_solvent[0] + N])
    dst2 = jnp.concatenate([edge_index_chromophore[1].reshape(NCH, CH),
                            edge_index_solvent[1].reshape(NCH, CH)])
    ones16 = jnp.ones((CH, 16), jnp.float32)
    z16 = jnp.zeros((N, 16), jnp.float32)
    z128 = jnp.zeros((N, 128), jnp.float32)
    z64 = jnp.zeros((N, 64), jnp.float32)

    degp = _sc_degree(dst2, ones16, z16)

    xall = jnp.concatenate([x_chromophore, x_solvent])
    h1 = _tc1(xall, jnp.stack([W1_c, W1_s]), degp)
    acc1 = _sc_conv128(h1, src, dst2, z128)
    y1, p1 = _tc_pre128(acc1, h1, degp)
    h2 = _tc2b(y1, p1, jnp.stack([g1_c, g1_s]), jnp.stack([be1_c, be1_s]),
               jnp.stack([W2_c, W2_s]), degp)

    acc2 = _sc_conv64(h2, src, dst2, z64)
    y2, p2 = _tc_pre64(acc2, h2, degp)
    pool, cnt = _tc3b(y2, p2, jnp.stack([g2_c, g2_s]),
                      jnp.stack([be2_c, be2_s]), batch.reshape(NBLK, 1, BLK))

    fc2_Wp = jnp.pad(fc2_W, ((0, 0), (0, 127)))
    fc2_bp = jnp.pad(fc2_b, (0, 127))
    out = _tc4(pool, cnt, fc1_W, gfc, befc, fc2_Wp, fc2_bp)
    return out[:, :1]


# speedup vs baseline: 33.3691x; 1.0133x over previous
import functools

import jax
import jax.numpy as jnp
from jax import lax
from jax.experimental import pallas as pl
from jax.experimental.pallas import tpu as pltpu
from jax.experimental.pallas import tpu_sc as plsc

N = 10000
E = 320000
NG = 256
EPS = 1e-5

NC = 2
NS = 16
CH = 80
EPT = E // NS
NCHT = EPT // CH
NCH = E // CH
RPT = N // NS

NBLK = 10
BLK = N // NBLK


_MESH = plsc.VectorSubcoreMesh(core_axis_name="c", subcore_axis_name="s")
_SC_PARAMS = pltpu.CompilerParams(use_tc_tiling_on_sc=False)


@functools.partial(
    pl.kernel,
    out_type=jax.ShapeDtypeStruct((2 * N, 16), jnp.float32),
    mesh=_MESH,
    compiler_params=_SC_PARAMS,
    scratch_types=[
        pltpu.VMEM((NCHT, CH), jnp.int32),
        pltpu.VMEM((CH, 16), jnp.float32),
        pltpu.SemaphoreType.DMA,
        pltpu.VMEM_SHARED((N, 16), jnp.float32),
    ],
)
def _sc_degree(dst_hbm, ones_hbm, zeros_hbm, out_hbm, dstb, onesb, sem, acc):
    cid = lax.axis_index("c")
    sid = lax.axis_index("s")
    pltpu.sync_copy(dst_hbm.at[pl.ds(cid * NCH + sid * NCHT, NCHT)], dstb)
    pltpu.sync_copy(ones_hbm, onesb)
    r0 = sid * RPT
    pltpu.sync_copy(zeros_hbm.at[pl.ds(r0, RPT)], acc.at[pl.ds(r0, RPT)])
    plsc.subcore_barrier()

    def fire(j, carry):
        pltpu.async_copy(onesb, acc.at[dstb.at[j]], sem, add=True)
        return carry

    lax.fori_loop(0, NCHT, fire, 0)

    def drain(j, carry):
        pltpu.make_async_copy(onesb, acc.at[dstb.at[j]], sem).wait()
        return carry

    lax.fori_loop(0, NCHT, drain, 0)
    plsc.subcore_barrier()
    pltpu.sync_copy(acc.at[pl.ds(r0, RPT)], out_hbm.at[pl.ds(cid * N + r0, RPT)])


def _make_sc_conv(D):
    @functools.partial(
        pl.kernel,
        out_type=jax.ShapeDtypeStruct((2 * N, D), jnp.float32),
        mesh=_MESH,
        compiler_params=_SC_PARAMS,
        scratch_types=[
            pltpu.VMEM((EPT // 2,), jnp.int32),
            pltpu.VMEM((NCHT // 2, CH), jnp.int32),
            pltpu.VMEM((CH, D), jnp.float32),
            pltpu.VMEM((CH, D), jnp.float32),
            pltpu.VMEM((CH, D), jnp.float32),
            pltpu.SemaphoreType.DMA,
            pltpu.SemaphoreType.DMA,
            pltpu.SemaphoreType.DMA,
            pltpu.SemaphoreType.DMA,
            pltpu.SemaphoreType.DMA,
            pltpu.SemaphoreType.DMA,
            pltpu.VMEM_SHARED((N, D), jnp.float32),
        ],
    )
    def conv(h_hbm, src_hbm, dst_hbm, zeros_hbm, out_hbm, srcb, dstb,
             b0, b1, b2, g0, g1, g2, s0, s1, s2, acc):
        cid = lax.axis_index("c")
        sid = lax.axis_index("s")
        r0 = sid * RPT
        pltpu.sync_copy(zeros_hbm.at[pl.ds(r0, RPT)], acc.at[pl.ds(r0, RPT)])
        plsc.subcore_barrier()
        bufs = (b0, b1, b2)
        gsems = (g0, g1, g2)
        ssems = (s0, s1, s2)

        def g_start(j, k):
            pltpu.async_copy(h_hbm.at[srcb.at[pl.ds(j * CH, CH)]],
                             bufs[k], gsems[k])

        def g_wait(j, k):
            pltpu.make_async_copy(h_hbm.at[srcb.at[pl.ds(j * CH, CH)]],
                                  bufs[k], gsems[k]).wait()

        def s_start(j, k):
            pltpu.async_copy(bufs[k], acc.at[dstb.at[j]], ssems[k], add=True)

        def s_drain(j, k):
            pltpu.make_async_copy(bufs[k], acc.at[dstb.at[j]],
                                  ssems[k]).wait()

        nh = NCHT // 2
        nloop = (nh - 2) // 3
        for h in range(2):
            pltpu.sync_copy(
                src_hbm.at[pl.ds(cid * E + sid * EPT + h * (EPT // 2),
                                 EPT // 2)], srcb)
            pltpu.sync_copy(
                dst_hbm.at[pl.ds(cid * NCH + sid * NCHT + h * nh, nh)], dstb)
            for k in range(3):
                g_start(k, k)

            def step(t, carry):
                for k in range(3):
                    j = 3 * t + k
                    g_wait(j, k)
                    s_start(j, k)

                    @pl.when(j + 3 < nh)
                    def _(j=j, k=k):
                        s_drain(j, k)
                        g_start(j + 3, k)
                return carry

            lax.fori_loop(0, nloop, step, 0)
            jt = nh - 2
            g_wait(jt, 0)
            s_start(jt, 0)
            g_wait(jt + 1, 1)
            s_start(jt + 1, 1)
            s_drain(jt - 1, 2)
            s_drain(jt, 0)
            s_drain(jt + 1, 1)
        plsc.subcore_barrier()
        pltpu.sync_copy(acc.at[pl.ds(r0, RPT)],
                        out_hbm.at[pl.ds(cid * N + r0, RPT)])

    return conv


_sc_conv128 = _make_sc_conv(128)
_sc_conv64 = _make_sc_conv(64)


NSTEP = 2 * NBLK


def _row_spec(d):
    return pl.BlockSpec((BLK, d), lambda i: (i, 0))


def _br_spec(shape):
    nd = len(shape)
    return pl.BlockSpec((1,) + shape, lambda i, _n=nd: (i // NBLK,) + (0,) * _n)


def _full_spec(shape):
    nd = len(shape)
    return pl.BlockSpec(shape, lambda i, _n=nd: (0,) * _n)


def _dinv(deg_blk):
    return lax.rsqrt(deg_blk[:, :1] + 1.0)


def _branch_sums(part_full):
    b = pl.program_id(0) // NBLK
    sel = (lax.broadcasted_iota(jnp.int32, (NSTEP, 1, 1), 0) // NBLK == b)
    return jnp.sum(jnp.where(sel, part_full, 0.0), axis=0)


def _sel_row(v2d):
    b = pl.program_id(0) // NBLK
    sel = (lax.broadcasted_iota(jnp.int32, (2, 1), 0) == b)
    return jnp.sum(jnp.where(sel, v2d, 0.0), axis=0)


def _tc1_body(x, w1, dg, h_o):
    h_o[...] = jnp.dot(x[...], w1[...][0],
                       preferred_element_type=jnp.float32) * _dinv(dg[...])


def _tc1(xall, W1, degp):
    return pl.pallas_call(
        _tc1_body,
        grid=(NSTEP,),
        in_specs=[_row_spec(128), _br_spec((128, 128)), _row_spec(16)],
        out_specs=_row_spec(128),
        out_shape=jax.ShapeDtypeStruct((2 * N, 128), jnp.float32),
    )(xall, W1, degp)


def _bn_apply(y, sums, g, be):
    mean = sums[0] / N
    var = sums[1] / N - mean * mean
    return (y - mean[None, :]) * lax.rsqrt(var[None, :] + EPS) * g[None, :] \
        + be[None, :]


def _accum_stats(y, sums_ref):
    b = pl.program_id(1) // NBLK

    @pl.when((pl.program_id(0) == 0) & (pl.program_id(1) == 0))
    def _():
        sums_ref[...] = jnp.zeros_like(sums_ref)

    upd = jnp.stack([jnp.sum(y, 0), jnp.sum(y * y, 0)])
    sel = (lax.broadcasted_iota(jnp.int32, (2, 1, 1), 0) == b)
    sums_ref[...] = sums_ref[...] + jnp.where(sel, upd[None], 0.0)


def _sel_branch_stats(sums_ref):
    b = pl.program_id(1) // NBLK
    sel = (lax.broadcasted_iota(jnp.int32, (2, 1, 1), 0) == b)
    return jnp.sum(jnp.where(sel, sums_ref[...], 0.0), axis=0)


def _layer1_body(ac, hc, dg, g1, be1, w2, h_o, ysc, sums):
    p = pl.program_id(0)

    @pl.when(p == 0)
    def _():
        i = pl.program_id(1)
        y = _dinv(dg[...]) * (ac[...] + hc[...])
        ysc[pl.ds(i * BLK, BLK), :] = y
        _accum_stats(y, sums)

    @pl.when(p == 1)
    def _():
        i = pl.program_id(1)
        y = ysc[pl.ds(i * BLK, BLK), :]
        x = jnp.maximum(_bn_apply(y, _sel_branch_stats(sums),
                                  _sel_row(g1[...]), _sel_row(be1[...])), 0.0)
        h_o[...] = jnp.dot(x, w2[...][0],
                           preferred_element_type=jnp.float32) * _dinv(dg[...])


def _tc_layer1(acc, h, degp, g1, be1, W2):
    return pl.pallas_call(
        _layer1_body,
        grid=(2, NSTEP),
        in_specs=[
            pl.BlockSpec((BLK, 128), lambda p, i: (i * (1 - p), 0)),
            pl.BlockSpec((BLK, 128), lambda p, i: (i * (1 - p), 0)),
            pl.BlockSpec((BLK, 16), lambda p, i: (i, 0)),
            pl.BlockSpec((2, 128), lambda p, i: (0, 0)),
            pl.BlockSpec((2, 128), lambda p, i: (0, 0)),
            pl.BlockSpec((1, 128, 64), lambda p, i: (i // NBLK, 0, 0)),
        ],
        out_specs=pl.BlockSpec((BLK, 64), lambda p, i: (i * p, 0)),
        out_shape=jax.ShapeDtypeStruct((2 * N, 64), jnp.float32),
        scratch_shapes=[pltpu.VMEM((2 * N, 128), jnp.float32),
                        pltpu.VMEM((2, 2, 128), jnp.float32)],
    )(acc, h, degp, g1, be1, W2)


def _layer2_body(ac, hc, dg, g2, be2, batch, pool_o, cnt_o, ysc, sums):
    p = pl.program_id(0)

    @pl.when(p == 0)
    def _():
        i = pl.program_id(1)
        y = _dinv(dg[...]) * (ac[...] + hc[...])
        ysc[pl.ds(i * BLK, BLK), :] = y
        _accum_stats(y, sums)

    @pl.when(p == 1)
    def _():
        i = pl.program_id(1)
        y = ysc[pl.ds(i * BLK, BLK), :]
        z = jnp.maximum(_bn_apply(y, _sel_branch_stats(sums),
                                  _sel_row(g2[...]), _sel_row(be2[...])), 0.0)
        gids = lax.broadcasted_iota(jnp.int32, (NG, BLK), 0)
        onehot = (batch[...][0] == gids).astype(jnp.float32)
        pool_o[...] = jnp.dot(onehot, z,
                              preferred_element_type=jnp.float32)[None]
        cnt_o[...] = jnp.broadcast_to(
            jnp.sum(onehot, axis=1, keepdims=True), (NG, 16))[None]


def _tc_layer2(acc, h, degp, g2, be2, batch3d):
    return pl.pallas_call(
        _layer2_body,
        grid=(2, NSTEP),
        in_specs=[
            pl.BlockSpec((BLK, 64), lambda p, i: (i * (1 - p), 0)),
            pl.BlockSpec((BLK, 64), lambda p, i: (i * (1 - p), 0)),
            pl.BlockSpec((BLK, 16), lambda p, i: (i, 0)),
            pl.BlockSpec((2, 64), lambda p, i: (0, 0)),
            pl.BlockSpec((2, 64), lambda p, i: (0, 0)),
            pl.BlockSpec((1, 1, BLK), lambda p, i: ((i % NBLK) * p, 0, 0)),
        ],
        out_specs=[pl.BlockSpec((1, NG, 64), lambda p, i: (i * p, 0, 0)),
                   pl.BlockSpec((1, NG, 16),
                                lambda p, i: ((i % NBLK) * p, 0, 0))],
        out_shape=[jax.ShapeDtypeStruct((NSTEP, NG, 64), jnp.float32),
                   jax.ShapeDtypeStruct((NBLK, NG, 16), jnp.float32)],
        scratch_shapes=[pltpu.VMEM((2 * N, 64), jnp.float32),
                        pltpu.VMEM((2, 2, 64), jnp.float32)],
    )(acc, h, degp, g2, be2, batch3d)


def _tc4_body(pool, cnt, fc1w, gfc, befc, fc2w, fc2b, out_o):
    pall = pool[...]
    c = jnp.maximum(jnp.sum(cnt[...], axis=0)[:, :1], 1.0)
    x = jnp.concatenate([jnp.sum(pall[:NBLK], 0) / c,
                         jnp.sum(pall[NBLK:], 0) / c], axis=1)
    h = jnp.dot(x, fc1w[...], preferred_element_type=jnp.float32)
    m = jnp.mean(h, axis=0)
    v = jnp.mean((h - m[None, :]) ** 2, axis=0)
    h = (h - m[None, :]) * lax.rsqrt(v[None, :] + EPS) * gfc[...][None, :] \
        + befc[...][None, :]
    h = jnp.maximum(h, 0.0)
    out_o[...] = jnp.dot(h, fc2w[...],
                         preferred_element_type=jnp.float32) + fc2b[...][None, :]


def _tc4(pool, cnt, fc1_W, gfc, befc, fc2_Wp, fc2_bp):
    return pl.pallas_call(
        _tc4_body,
        grid=(1,),
        in_specs=[_full_spec((NSTEP, NG, 64)), _full_spec((NBLK, NG, 16)),
                  _full_spec((128, 64)), _full_spec((64,)), _full_spec((64,)),
                  _full_spec((64, 128)), _full_spec((128,))],
        out_specs=_full_spec((NG, 128)),
        out_shape=jax.ShapeDtypeStruct((NG, 128), jnp.float32),
    )(pool, cnt, fc1_W, gfc, befc, fc2_Wp, fc2_bp)


def kernel(x_chromophore, edge_index_chromophore, edge_attr_chromophore,
           x_solvent, edge_index_solvent, edge_attr_solvent, batch,
           W1_c, b1_c, g1_c, be1_c, W2_c, b2_c, g2_c, be2_c,
           W1_s, b1_s, g1_s, be1_s, W2_s, b2_s, g2_s, be2_s,
           fc1_W, fc1_b, gfc, befc, fc2_W, fc2_b):
    del edge_attr_chromophore, edge_attr_solvent
    del b1_c, b2_c, b1_s, b2_s, fc1_b

    src = jnp.concatenate([edge_index_chromophore[0],
                           edge_index_solvent[0] + N])
    dst2 = jnp.concatenate([edge_index_chromophore[1].reshape(NCH, CH),
                            edge_index_solvent[1].reshape(NCH, CH)])
    ones16 = jnp.ones((CH, 16), jnp.float32)
    z16 = jnp.zeros((N, 16), jnp.float32)
    z128 = jnp.zeros((N, 128), jnp.float32)
    z64 = jnp.zeros((N, 64), jnp.float32)

    degp = _sc_degree(dst2, ones16, z16)

    xall = jnp.concatenate([x_chromophore, x_solvent])
    h1 = _tc1(xall, jnp.stack([W1_c, W1_s]), degp)
    acc1 = _sc_conv128(h1, src, dst2, z128)
    h2 = _tc_layer1(acc1, h1, degp, jnp.stack([g1_c, g1_s]),
                    jnp.stack([be1_c, be1_s]), jnp.stack([W2_c, W2_s]))

    acc2 = _sc_conv64(h2, src, dst2, z64)
    pool, cnt = _tc_layer2(acc2, h2, degp, jnp.stack([g2_c, g2_s]),
                           jnp.stack([be2_c, be2_s]),
                           batch.reshape(NBLK, 1, BLK))

    fc2_Wp = jnp.pad(fc2_W, ((0, 0), (0, 127)))
    fc2_bp = jnp.pad(fc2_b, (0, 127))
    out = _tc4(pool, cnt, fc1_W, gfc, befc, fc2_Wp, fc2_bp)
    return out[:, :1]


# speedup vs baseline: 33.4546x; 1.0026x over previous
import functools

import jax
import jax.numpy as jnp
from jax import lax
from jax.experimental import pallas as pl
from jax.experimental.pallas import tpu as pltpu
from jax.experimental.pallas import tpu_sc as plsc

N = 10000
E = 320000
NG = 256
EPS = 1e-5

NC = 2
NS = 16
CH = 80
EPT = E // NS
NCHT = EPT // CH
NCH = E // CH
RPT = N // NS

NBLK = 10
BLK = N // NBLK


_MESH = plsc.VectorSubcoreMesh(core_axis_name="c", subcore_axis_name="s")
_SC_PARAMS = pltpu.CompilerParams(use_tc_tiling_on_sc=False)


@functools.partial(
    pl.kernel,
    out_type=jax.ShapeDtypeStruct((2 * N, 16), jnp.float32),
    mesh=_MESH,
    compiler_params=_SC_PARAMS,
    scratch_types=[
        pltpu.VMEM((NCHT, CH), jnp.int32),
        pltpu.VMEM((CH, 16), jnp.float32),
        pltpu.SemaphoreType.DMA,
        pltpu.VMEM_SHARED((N, 16), jnp.float32),
    ],
)
def _sc_degree(dstc_hbm, dsts_hbm, ones_hbm, zeros_hbm, out_hbm,
               dstb, onesb, sem, acc):
    cid = lax.axis_index("c")
    sid = lax.axis_index("s")

    @pl.when(cid == 0)
    def _():
        pltpu.sync_copy(dstc_hbm.at[pl.ds(sid * NCHT, NCHT)], dstb)

    @pl.when(cid == 1)
    def _():
        pltpu.sync_copy(dsts_hbm.at[pl.ds(sid * NCHT, NCHT)], dstb)

    pltpu.sync_copy(ones_hbm, onesb)
    r0 = sid * RPT
    pltpu.sync_copy(zeros_hbm.at[pl.ds(r0, RPT)], acc.at[pl.ds(r0, RPT)])
    plsc.subcore_barrier()

    def fire(j, carry):
        pltpu.async_copy(onesb, acc.at[dstb.at[j]], sem, add=True)
        return carry

    lax.fori_loop(0, NCHT, fire, 0)

    def drain(j, carry):
        pltpu.make_async_copy(onesb, acc.at[dstb.at[j]], sem).wait()
        return carry

    lax.fori_loop(0, NCHT, drain, 0)
    plsc.subcore_barrier()
    pltpu.sync_copy(acc.at[pl.ds(r0, RPT)], out_hbm.at[pl.ds(cid * N + r0, RPT)])


def _make_sc_conv(D):
    @functools.partial(
        pl.kernel,
        out_type=jax.ShapeDtypeStruct((2 * N, D), jnp.float32),
        mesh=_MESH,
        compiler_params=_SC_PARAMS,
        scratch_types=[
            pltpu.VMEM((EPT // 2,), jnp.int32),
            pltpu.VMEM((NCHT // 2, CH), jnp.int32),
            pltpu.VMEM((CH, D), jnp.float32),
            pltpu.VMEM((CH, D), jnp.float32),
            pltpu.VMEM((CH, D), jnp.float32),
            pltpu.SemaphoreType.DMA,
            pltpu.SemaphoreType.DMA,
            pltpu.SemaphoreType.DMA,
            pltpu.SemaphoreType.DMA,
            pltpu.SemaphoreType.DMA,
            pltpu.SemaphoreType.DMA,
            pltpu.VMEM_SHARED((N, D), jnp.float32),
        ],
    )
    def conv(h_hbm, srcc_hbm, srcs_hbm, dstc_hbm, dsts_hbm, zeros_hbm,
             out_hbm, srcb, dstb, b0, b1, b2, g0, g1, g2, s0, s1, s2, acc):
        cid = lax.axis_index("c")
        sid = lax.axis_index("s")
        r0 = sid * RPT
        pltpu.sync_copy(zeros_hbm.at[pl.ds(r0, RPT)], acc.at[pl.ds(r0, RPT)])
        plsc.subcore_barrier()
        bufs = (b0, b1, b2)
        gsems = (g0, g1, g2)
        ssems = (s0, s1, s2)
        table = h_hbm.at[pl.ds(cid * N, N)]

        def g_start(j, k):
            pltpu.async_copy(table.at[srcb.at[pl.ds(j * CH, CH)]],
                             bufs[k], gsems[k])

        def g_wait(j, k):
            pltpu.make_async_copy(table.at[srcb.at[pl.ds(j * CH, CH)]],
                                  bufs[k], gsems[k]).wait()

        def s_start(j, k):
            pltpu.async_copy(bufs[k], acc.at[dstb.at[j]], ssems[k], add=True)

        def s_drain(j, k):
            pltpu.make_async_copy(bufs[k], acc.at[dstb.at[j]],
                                  ssems[k]).wait()

        nh = NCHT // 2
        nloop = (nh - 2) // 3
        for h in range(2):
            @pl.when(cid == 0)
            def _(h=h):
                pltpu.sync_copy(
                    srcc_hbm.at[pl.ds(sid * EPT + h * (EPT // 2),
                                      EPT // 2)], srcb)
                pltpu.sync_copy(
                    dstc_hbm.at[pl.ds(sid * NCHT + h * nh, nh)], dstb)

            @pl.when(cid == 1)
            def _(h=h):
                pltpu.sync_copy(
                    srcs_hbm.at[pl.ds(sid * EPT + h * (EPT // 2),
                                      EPT // 2)], srcb)
                pltpu.sync_copy(
                    dsts_hbm.at[pl.ds(sid * NCHT + h * nh, nh)], dstb)
            for k in range(3):
                g_start(k, k)

            def step(t, carry):
                for k in range(3):
                    j = 3 * t + k
                    g_wait(j, k)
                    s_start(j, k)

                    @pl.when(j + 3 < nh)
                    def _(j=j, k=k):
                        s_drain(j, k)
                        g_start(j + 3, k)
                return carry

            lax.fori_loop(0, nloop, step, 0)
            jt = nh - 2
            g_wait(jt, 0)
            s_start(jt, 0)
            g_wait(jt + 1, 1)
            s_start(jt + 1, 1)
            s_drain(jt - 1, 2)
            s_drain(jt, 0)
            s_drain(jt + 1, 1)
        plsc.subcore_barrier()
        pltpu.sync_copy(acc.at[pl.ds(r0, RPT)],
                        out_hbm.at[pl.ds(cid * N + r0, RPT)])

    return conv


_sc_conv128 = _make_sc_conv(128)
_sc_conv64 = _make_sc_conv(64)


NSTEP = 2 * NBLK


def _row_spec(d):
    return pl.BlockSpec((BLK, d), lambda i: (i, 0))


def _br_spec(shape):
    nd = len(shape)
    return pl.BlockSpec((1,) + shape, lambda i, _n=nd: (i // NBLK,) + (0,) * _n)


def _full_spec(shape):
    nd = len(shape)
    return pl.BlockSpec(shape, lambda i, _n=nd: (0,) * _n)


def _dinv(deg_blk):
    return lax.rsqrt(deg_blk[:, :1] + 1.0)


def _branch_sums(part_full):
    b = pl.program_id(0) // NBLK
    sel = (lax.broadcasted_iota(jnp.int32, (NSTEP, 1, 1), 0) // NBLK == b)
    return jnp.sum(jnp.where(sel, part_full, 0.0), axis=0)


def _sel_row(v2d):
    b = pl.program_id(0) // NBLK
    sel = (lax.broadcasted_iota(jnp.int32, (2, 1), 0) == b)
    return jnp.sum(jnp.where(sel, v2d, 0.0), axis=0)


def _tc1_body(xc, xs, w1, dg, h_o):
    b = pl.program_id(0) // NBLK
    x = jnp.where(b == 0, xc[...], xs[...])
    h_o[...] = jnp.dot(x, w1[...][0],
                       preferred_element_type=jnp.float32) * _dinv(dg[...])


def _tc1(x_c, x_s, W1, degp):
    return pl.pallas_call(
        _tc1_body,
        grid=(NSTEP,),
        in_specs=[pl.BlockSpec((BLK, 128), lambda i: (i % NBLK, 0)),
                  pl.BlockSpec((BLK, 128), lambda i: (i % NBLK, 0)),
                  _br_spec((128, 128)), _row_spec(16)],
        out_specs=_row_spec(128),
        out_shape=jax.ShapeDtypeStruct((2 * N, 128), jnp.float32),
    )(x_c, x_s, W1, degp)


def _bn_apply(y, sums, g, be):
    mean = sums[0] / N
    var = sums[1] / N - mean * mean
    return (y - mean[None, :]) * lax.rsqrt(var[None, :] + EPS) * g[None, :] \
        + be[None, :]


def _accum_stats(y, sums_ref):
    b = pl.program_id(1) // NBLK

    @pl.when((pl.program_id(0) == 0) & (pl.program_id(1) == 0))
    def _():
        sums_ref[...] = jnp.zeros_like(sums_ref)

    upd = jnp.stack([jnp.sum(y, 0), jnp.sum(y * y, 0)])
    sel = (lax.broadcasted_iota(jnp.int32, (2, 1, 1), 0) == b)
    sums_ref[...] = sums_ref[...] + jnp.where(sel, upd[None], 0.0)


def _sel_branch_stats(sums_ref):
    b = pl.program_id(1) // NBLK
    sel = (lax.broadcasted_iota(jnp.int32, (2, 1, 1), 0) == b)
    return jnp.sum(jnp.where(sel, sums_ref[...], 0.0), axis=0)


def _layer1_body(ac, hc, dg, g1, be1, w2, h_o, ysc, sums):
    p = pl.program_id(0)

    @pl.when(p == 0)
    def _():
        i = pl.program_id(1)
        y = _dinv(dg[...]) * (ac[...] + hc[...])
        ysc[pl.ds(i * BLK, BLK), :] = y
        _accum_stats(y, sums)

    @pl.when(p == 1)
    def _():
        i = pl.program_id(1)
        y = ysc[pl.ds(i * BLK, BLK), :]
        x = jnp.maximum(_bn_apply(y, _sel_branch_stats(sums),
                                  _sel_row(g1[...]), _sel_row(be1[...])), 0.0)
        h_o[...] = jnp.dot(x, w2[...][0],
                           preferred_element_type=jnp.float32) * _dinv(dg[...])


def _tc_layer1(acc, h, degp, g1, be1, W2):
    return pl.pallas_call(
        _layer1_body,
        grid=(2, NSTEP),
        in_specs=[
            pl.BlockSpec((BLK, 128), lambda p, i: (i * (1 - p), 0)),
            pl.BlockSpec((BLK, 128), lambda p, i: (i * (1 - p), 0)),
            pl.BlockSpec((BLK, 16), lambda p, i: (i, 0)),
            pl.BlockSpec((2, 128), lambda p, i: (0, 0)),
            pl.BlockSpec((2, 128), lambda p, i: (0, 0)),
            pl.BlockSpec((1, 128, 64), lambda p, i: (i // NBLK, 0, 0)),
        ],
        out_specs=pl.BlockSpec((BLK, 64), lambda p, i: (i * p, 0)),
        out_shape=jax.ShapeDtypeStruct((2 * N, 64), jnp.float32),
        scratch_shapes=[pltpu.VMEM((2 * N, 128), jnp.float32),
                        pltpu.VMEM((2, 2, 128), jnp.float32)],
    )(acc, h, degp, g1, be1, W2)


def _layer2_body(ac, hc, dg, g2, be2, batch, pool_o, cnt_o, ysc, sums):
    p = pl.program_id(0)

    @pl.when(p == 0)
    def _():
        i = pl.program_id(1)
        y = _dinv(dg[...]) * (ac[...] + hc[...])
        ysc[pl.ds(i * BLK, BLK), :] = y
        _accum_stats(y, sums)

    @pl.when(p == 1)
    def _():
        i = pl.program_id(1)
        y = ysc[pl.ds(i * BLK, BLK), :]
        z = jnp.maximum(_bn_apply(y, _sel_branch_stats(sums),
                                  _sel_row(g2[...]), _sel_row(be2[...])), 0.0)
        gids = lax.broadcasted_iota(jnp.int32, (NG, BLK), 0)
        onehot = (batch[...][0] == gids).astype(jnp.float32)
        pool_o[...] = jnp.dot(onehot, z,
                              preferred_element_type=jnp.float32)[None]
        cnt_o[...] = jnp.broadcast_to(
            jnp.sum(onehot, axis=1, keepdims=True), (NG, 16))[None]


def _tc_layer2(acc, h, degp, g2, be2, batch3d):
    return pl.pallas_call(
        _layer2_body,
        grid=(2, NSTEP),
        in_specs=[
            pl.BlockSpec((BLK, 64), lambda p, i: (i * (1 - p), 0)),
            pl.BlockSpec((BLK, 64), lambda p, i: (i * (1 - p), 0)),
            pl.BlockSpec((BLK, 16), lambda p, i: (i, 0)),
            pl.BlockSpec((2, 64), lambda p, i: (0, 0)),
            pl.BlockSpec((2, 64), lambda p, i: (0, 0)),
            pl.BlockSpec((1, 1, BLK), lambda p, i: ((i % NBLK) * p, 0, 0)),
        ],
        out_specs=[pl.BlockSpec((1, NG, 64), lambda p, i: (i * p, 0, 0)),
                   pl.BlockSpec((1, NG, 16),
                                lambda p, i: ((i % NBLK) * p, 0, 0))],
        out_shape=[jax.ShapeDtypeStruct((NSTEP, NG, 64), jnp.float32),
                   jax.ShapeDtypeStruct((NBLK, NG, 16), jnp.float32)],
        scratch_shapes=[pltpu.VMEM((2 * N, 64), jnp.float32),
                        pltpu.VMEM((2, 2, 64), jnp.float32)],
    )(acc, h, degp, g2, be2, batch3d)


def _tc4_body(pool, cnt, fc1w, gfc, befc, fc2w, fc2b, out_o):
    pall = pool[...]
    c = jnp.maximum(jnp.sum(cnt[...], axis=0)[:, :1], 1.0)
    x = jnp.concatenate([jnp.sum(pall[:NBLK], 0) / c,
                         jnp.sum(pall[NBLK:], 0) / c], axis=1)
    h = jnp.dot(x, fc1w[...], preferred_element_type=jnp.float32)
    m = jnp.mean(h, axis=0)
    v = jnp.mean((h - m[None, :]) ** 2, axis=0)
    h = (h - m[None, :]) * lax.rsqrt(v[None, :] + EPS) * gfc[...][None, :] \
        + befc[...][None, :]
    h = jnp.maximum(h, 0.0)
    out_o[...] = jnp.dot(h, fc2w[...],
                         preferred_element_type=jnp.float32) + fc2b[...][None, :]


def _tc4(pool, cnt, fc1_W, gfc, befc, fc2_Wp, fc2_bp):
    return pl.pallas_call(
        _tc4_body,
        grid=(1,),
        in_specs=[_full_spec((NSTEP, NG, 64)), _full_spec((NBLK, NG, 16)),
                  _full_spec((128, 64)), _full_spec((64,)), _full_spec((64,)),
                  _full_spec((64, 128)), _full_spec((128,))],
        out_specs=_full_spec((NG, 128)),
        out_shape=jax.ShapeDtypeStruct((NG, 128), jnp.float32),
    )(pool, cnt, fc1_W, gfc, befc, fc2_Wp, fc2_bp)


def kernel(x_chromophore, edge_index_chromophore, edge_attr_chromophore,
           x_solvent, edge_index_solvent, edge_attr_solvent, batch,
           W1_c, b1_c, g1_c, be1_c, W2_c, b2_c, g2_c, be2_c,
           W1_s, b1_s, g1_s, be1_s, W2_s, b2_s, g2_s, be2_s,
           fc1_W, fc1_b, gfc, befc, fc2_W, fc2_b):
    del edge_attr_chromophore, edge_attr_solvent
    del b1_c, b2_c, b1_s, b2_s, fc1_b

    src_c = edge_index_chromophore[0]
    src_s = edge_index_solvent[0]
    dst2_c = edge_index_chromophore[1].reshape(NCH, CH)
    dst2_s = edge_index_solvent[1].reshape(NCH, CH)
    ones16 = jnp.ones((CH, 16), jnp.float32)
    z16 = jnp.zeros((N, 16), jnp.float32)
    z128 = jnp.zeros((N, 128), jnp.float32)
    z64 = jnp.zeros((N, 64), jnp.float32)

    degp = _sc_degree(dst2_c, dst2_s, ones16, z16)

    h1 = _tc1(x_chromophore, x_solvent, jnp.stack([W1_c, W1_s]), degp)
    acc1 = _sc_conv128(h1, src_c, src_s, dst2_c, dst2_s, z128)
    h2 = _tc_layer1(acc1, h1, degp, jnp.stack([g1_c, g1_s]),
                    jnp.stack([be1_c, be1_s]), jnp.stack([W2_c, W2_s]))

    acc2 = _sc_conv64(h2, src_c, src_s, dst2_c, dst2_s, z64)
    pool, cnt = _tc_layer2(acc2, h2, degp, jnp.stack([g2_c, g2_s]),
                           jnp.stack([be2_c, be2_s]),
                           batch.reshape(NBLK, 1, BLK))

    fc2_Wp = jnp.pad(fc2_W, ((0, 0), (0, 127)))
    fc2_bp = jnp.pad(fc2_b, (0, 127))
    out = _tc4(pool, cnt, fc1_W, gfc, befc, fc2_Wp, fc2_bp)
    return out[:, :1]


# speedup vs baseline: 36.6342x; 1.0950x over previous
import functools

import jax
import jax.numpy as jnp
from jax import lax
from jax.experimental import pallas as pl
from jax.experimental.pallas import tpu as pltpu
from jax.experimental.pallas import tpu_sc as plsc

N = 10000
E = 320000
NG = 256
EPS = 1e-5

NC = 2
NS = 16
CH = 80
EPT = E // NS
NCHT = EPT // CH
NCH = E // CH
RPT = N // NS

NBLK = 2
BLK = N // NBLK


_MESH = plsc.VectorSubcoreMesh(core_axis_name="c", subcore_axis_name="s")
_SC_PARAMS = pltpu.CompilerParams(use_tc_tiling_on_sc=False)


@functools.partial(
    pl.kernel,
    out_type=jax.ShapeDtypeStruct((2 * N, 16), jnp.float32),
    mesh=_MESH,
    compiler_params=_SC_PARAMS,
    scratch_types=[
        pltpu.VMEM((NCHT, CH), jnp.int32),
        pltpu.VMEM((CH, 16), jnp.float32),
        pltpu.SemaphoreType.DMA,
        pltpu.VMEM_SHARED((N, 16), jnp.float32),
    ],
)
def _sc_degree(dstc_hbm, dsts_hbm, ones_hbm, zeros_hbm, out_hbm,
               dstb, onesb, sem, acc):
    cid = lax.axis_index("c")
    sid = lax.axis_index("s")

    @pl.when(cid == 0)
    def _():
        pltpu.sync_copy(dstc_hbm.at[pl.ds(sid * NCHT, NCHT)], dstb)

    @pl.when(cid == 1)
    def _():
        pltpu.sync_copy(dsts_hbm.at[pl.ds(sid * NCHT, NCHT)], dstb)

    pltpu.sync_copy(ones_hbm, onesb)
    r0 = sid * RPT
    pltpu.sync_copy(zeros_hbm.at[pl.ds(r0, RPT)], acc.at[pl.ds(r0, RPT)])
    plsc.subcore_barrier()

    def fire(j, carry):
        pltpu.async_copy(onesb, acc.at[dstb.at[j]], sem, add=True)
        return carry

    lax.fori_loop(0, NCHT, fire, 0)

    def drain(j, carry):
        pltpu.make_async_copy(onesb, acc.at[dstb.at[j]], sem).wait()
        return carry

    lax.fori_loop(0, NCHT, drain, 0)
    plsc.subcore_barrier()
    pltpu.sync_copy(acc.at[pl.ds(r0, RPT)], out_hbm.at[pl.ds(cid * N + r0, RPT)])


def _make_sc_conv(D):
    @functools.partial(
        pl.kernel,
        out_type=jax.ShapeDtypeStruct((2 * N, D), jnp.float32),
        mesh=_MESH,
        compiler_params=_SC_PARAMS,
        scratch_types=[
            pltpu.VMEM((EPT // 2,), jnp.int32),
            pltpu.VMEM((NCHT // 2, CH), jnp.int32),
            pltpu.VMEM((CH, D), jnp.float32),
            pltpu.VMEM((CH, D), jnp.float32),
            pltpu.VMEM((CH, D), jnp.float32),
            pltpu.SemaphoreType.DMA,
            pltpu.SemaphoreType.DMA,
            pltpu.SemaphoreType.DMA,
            pltpu.SemaphoreType.DMA,
            pltpu.SemaphoreType.DMA,
            pltpu.SemaphoreType.DMA,
            pltpu.VMEM_SHARED((N, D), jnp.float32),
        ],
    )
    def conv(h_hbm, srcc_hbm, srcs_hbm, dstc_hbm, dsts_hbm, zeros_hbm,
             out_hbm, srcb, dstb, b0, b1, b2, g0, g1, g2, s0, s1, s2, acc):
        cid = lax.axis_index("c")
        sid = lax.axis_index("s")
        r0 = sid * RPT
        pltpu.sync_copy(zeros_hbm.at[pl.ds(r0, RPT)], acc.at[pl.ds(r0, RPT)])
        plsc.subcore_barrier()
        bufs = (b0, b1, b2)
        gsems = (g0, g1, g2)
        ssems = (s0, s1, s2)
        table = h_hbm.at[pl.ds(cid * N, N)]

        def g_start(j, k):
            pltpu.async_copy(table.at[srcb.at[pl.ds(j * CH, CH)]],
                             bufs[k], gsems[k])

        def g_wait(j, k):
            pltpu.make_async_copy(table.at[srcb.at[pl.ds(j * CH, CH)]],
                                  bufs[k], gsems[k]).wait()

        def s_start(j, k):
            pltpu.async_copy(bufs[k], acc.at[dstb.at[j]], ssems[k], add=True)

        def s_drain(j, k):
            pltpu.make_async_copy(bufs[k], acc.at[dstb.at[j]],
                                  ssems[k]).wait()

        nh = NCHT // 2
        nloop = (nh - 2) // 3
        for h in range(2):
            @pl.when(cid == 0)
            def _(h=h):
                pltpu.sync_copy(
                    srcc_hbm.at[pl.ds(sid * EPT + h * (EPT // 2),
                                      EPT // 2)], srcb)
                pltpu.sync_copy(
                    dstc_hbm.at[pl.ds(sid * NCHT + h * nh, nh)], dstb)

            @pl.when(cid == 1)
            def _(h=h):
                pltpu.sync_copy(
                    srcs_hbm.at[pl.ds(sid * EPT + h * (EPT // 2),
                                      EPT // 2)], srcb)
                pltpu.sync_copy(
                    dsts_hbm.at[pl.ds(sid * NCHT + h * nh, nh)], dstb)
            for k in range(3):
                g_start(k, k)

            def step(t, carry):
                for k in range(3):
                    j = 3 * t + k
                    g_wait(j, k)
                    s_start(j, k)

                    @pl.when(j + 3 < nh)
                    def _(j=j, k=k):
                        s_drain(j, k)
                        g_start(j + 3, k)
                return carry

            lax.fori_loop(0, nloop, step, 0)
            jt = nh - 2
            g_wait(jt, 0)
            s_start(jt, 0)
            g_wait(jt + 1, 1)
            s_start(jt + 1, 1)
            s_drain(jt - 1, 2)
            s_drain(jt, 0)
            s_drain(jt + 1, 1)
        plsc.subcore_barrier()
        pltpu.sync_copy(acc.at[pl.ds(r0, RPT)],
                        out_hbm.at[pl.ds(cid * N + r0, RPT)])

    return conv


_sc_conv128 = _make_sc_conv(128)
_sc_conv64 = _make_sc_conv(64)


NSTEP = 2 * NBLK


def _row_spec(d):
    return pl.BlockSpec((BLK, d), lambda i: (i, 0))


def _br_spec(shape):
    nd = len(shape)
    return pl.BlockSpec((1,) + shape, lambda i, _n=nd: (i // NBLK,) + (0,) * _n)


def _full_spec(shape):
    nd = len(shape)
    return pl.BlockSpec(shape, lambda i, _n=nd: (0,) * _n)


def _dinv(deg_blk):
    return lax.rsqrt(deg_blk[:, :1] + 1.0)


def _branch_sums(part_full):
    b = pl.program_id(0) // NBLK
    sel = (lax.broadcasted_iota(jnp.int32, (NSTEP, 1, 1), 0) // NBLK == b)
    return jnp.sum(jnp.where(sel, part_full, 0.0), axis=0)


def _sel_row(v2d):
    b = pl.program_id(0) // NBLK
    sel = (lax.broadcasted_iota(jnp.int32, (2, 1), 0) == b)
    return jnp.sum(jnp.where(sel, v2d, 0.0), axis=0)


def _tc1_body(xc, xs, w1, dg, h_o):
    b = pl.program_id(0) // NBLK
    x = jnp.where(b == 0, xc[...], xs[...])
    h_o[...] = jnp.dot(x, w1[...][0],
                       preferred_element_type=jnp.float32) * _dinv(dg[...])


def _tc1(x_c, x_s, W1, degp):
    return pl.pallas_call(
        _tc1_body,
        grid=(NSTEP,),
        in_specs=[pl.BlockSpec((BLK, 128), lambda i: (i % NBLK, 0)),
                  pl.BlockSpec((BLK, 128), lambda i: (i % NBLK, 0)),
                  _br_spec((128, 128)), _row_spec(16)],
        out_specs=_row_spec(128),
        out_shape=jax.ShapeDtypeStruct((2 * N, 128), jnp.float32),
    )(x_c, x_s, W1, degp)


def _bn_apply(y, sums, g, be):
    mean = sums[0] / N
    var = sums[1] / N - mean * mean
    return (y - mean[None, :]) * lax.rsqrt(var[None, :] + EPS) * g[None, :] \
        + be[None, :]


def _accum_stats(y, sums_ref):
    b = pl.program_id(1) // NBLK

    @pl.when((pl.program_id(0) == 0) & (pl.program_id(1) == 0))
    def _():
        sums_ref[...] = jnp.zeros_like(sums_ref)

    upd = jnp.stack([jnp.sum(y, 0), jnp.sum(y * y, 0)])
    sel = (lax.broadcasted_iota(jnp.int32, (2, 1, 1), 0) == b)
    sums_ref[...] = sums_ref[...] + jnp.where(sel, upd[None], 0.0)


def _sel_branch_stats(sums_ref):
    b = pl.program_id(1) // NBLK
    sel = (lax.broadcasted_iota(jnp.int32, (2, 1, 1), 0) == b)
    return jnp.sum(jnp.where(sel, sums_ref[...], 0.0), axis=0)


def _layer1_body(ac, hc, dg, g1, be1, w2, h_o, ysc, sums):
    p = pl.program_id(0)

    @pl.when(p == 0)
    def _():
        i = pl.program_id(1)
        y = _dinv(dg[...]) * (ac[...] + hc[...])
        ysc[pl.ds(i * BLK, BLK), :] = y
        _accum_stats(y, sums)

    @pl.when(p == 1)
    def _():
        i = pl.program_id(1)
        y = ysc[pl.ds(i * BLK, BLK), :]
        x = jnp.maximum(_bn_apply(y, _sel_branch_stats(sums),
                                  _sel_row(g1[...]), _sel_row(be1[...])), 0.0)
        h_o[...] = jnp.dot(x, w2[...][0],
                           preferred_element_type=jnp.float32) * _dinv(dg[...])


def _tc_layer1(acc, h, degp, g1, be1, W2):
    return pl.pallas_call(
        _layer1_body,
        grid=(2, NSTEP),
        in_specs=[
            pl.BlockSpec((BLK, 128), lambda p, i: (i * (1 - p), 0)),
            pl.BlockSpec((BLK, 128), lambda p, i: (i * (1 - p), 0)),
            pl.BlockSpec((BLK, 16), lambda p, i: (i, 0)),
            pl.BlockSpec((2, 128), lambda p, i: (0, 0)),
            pl.BlockSpec((2, 128), lambda p, i: (0, 0)),
            pl.BlockSpec((1, 128, 64), lambda p, i: (i // NBLK, 0, 0)),
        ],
        out_specs=pl.BlockSpec((BLK, 64), lambda p, i: (i * p, 0)),
        out_shape=jax.ShapeDtypeStruct((2 * N, 64), jnp.float32),
        scratch_shapes=[pltpu.VMEM((2 * N, 128), jnp.float32),
                        pltpu.VMEM((2, 2, 128), jnp.float32)],
    )(acc, h, degp, g1, be1, W2)


def _layer2_body(ac, hc, dg, g2, be2, batch, pool_o, cnt_o, ysc, sums):
    p = pl.program_id(0)

    @pl.when(p == 0)
    def _():
        i = pl.program_id(1)
        y = _dinv(dg[...]) * (ac[...] + hc[...])
        ysc[pl.ds(i * BLK, BLK), :] = y
        _accum_stats(y, sums)

    @pl.when(p == 1)
    def _():
        i = pl.program_id(1)
        y = ysc[pl.ds(i * BLK, BLK), :]
        z = jnp.maximum(_bn_apply(y, _sel_branch_stats(sums),
                                  _sel_row(g2[...]), _sel_row(be2[...])), 0.0)
        gids = lax.broadcasted_iota(jnp.int32, (NG, BLK), 0)
        onehot = (batch[...][0] == gids).astype(jnp.float32)
        pool_o[...] = jnp.dot(onehot, z,
                              preferred_element_type=jnp.float32)[None]
        cnt_o[...] = jnp.broadcast_to(
            jnp.sum(onehot, axis=1, keepdims=True), (NG, 16))[None]


def _tc_layer2(acc, h, degp, g2, be2, batch3d):
    return pl.pallas_call(
        _layer2_body,
        grid=(2, NSTEP),
        in_specs=[
            pl.BlockSpec((BLK, 64), lambda p, i: (i * (1 - p), 0)),
            pl.BlockSpec((BLK, 64), lambda p, i: (i * (1 - p), 0)),
            pl.BlockSpec((BLK, 16), lambda p, i: (i, 0)),
            pl.BlockSpec((2, 64), lambda p, i: (0, 0)),
            pl.BlockSpec((2, 64), lambda p, i: (0, 0)),
            pl.BlockSpec((1, 1, BLK), lambda p, i: ((i % NBLK) * p, 0, 0)),
        ],
        out_specs=[pl.BlockSpec((1, NG, 64), lambda p, i: (i * p, 0, 0)),
                   pl.BlockSpec((1, NG, 16),
                                lambda p, i: ((i % NBLK) * p, 0, 0))],
        out_shape=[jax.ShapeDtypeStruct((NSTEP, NG, 64), jnp.float32),
                   jax.ShapeDtypeStruct((NBLK, NG, 16), jnp.float32)],
        scratch_shapes=[pltpu.VMEM((2 * N, 64), jnp.float32),
                        pltpu.VMEM((2, 2, 64), jnp.float32)],
    )(acc, h, degp, g2, be2, batch3d)


def _tc4_body(pool, cnt, fc1w, gfc, befc, fc2w, fc2b, out_o):
    pall = pool[...]
    c = jnp.maximum(jnp.sum(cnt[...], axis=0)[:, :1], 1.0)
    x = jnp.concatenate([jnp.sum(pall[:NBLK], 0) / c,
                         jnp.sum(pall[NBLK:], 0) / c], axis=1)
    h = jnp.dot(x, fc1w[...], preferred_element_type=jnp.float32)
    m = jnp.mean(h, axis=0)
    v = jnp.mean((h - m[None, :]) ** 2, axis=0)
    h = (h - m[None, :]) * lax.rsqrt(v[None, :] + EPS) * gfc[...][None, :] \
        + befc[...][None, :]
    h = jnp.maximum(h, 0.0)
    out_o[...] = jnp.dot(h, fc2w[...],
                         preferred_element_type=jnp.float32) + fc2b[...][None, :]


def _tc4(pool, cnt, fc1_W, gfc, befc, fc2_Wp, fc2_bp):
    return pl.pallas_call(
        _tc4_body,
        grid=(1,),
        in_specs=[_full_spec((NSTEP, NG, 64)), _full_spec((NBLK, NG, 16)),
                  _full_spec((128, 64)), _full_spec((64,)), _full_spec((64,)),
                  _full_spec((64, 128)), _full_spec((128,))],
        out_specs=_full_spec((NG, 128)),
        out_shape=jax.ShapeDtypeStruct((NG, 128), jnp.float32),
    )(pool, cnt, fc1_W, gfc, befc, fc2_Wp, fc2_bp)


def kernel(x_chromophore, edge_index_chromophore, edge_attr_chromophore,
           x_solvent, edge_index_solvent, edge_attr_solvent, batch,
           W1_c, b1_c, g1_c, be1_c, W2_c, b2_c, g2_c, be2_c,
           W1_s, b1_s, g1_s, be1_s, W2_s, b2_s, g2_s, be2_s,
           fc1_W, fc1_b, gfc, befc, fc2_W, fc2_b):
    del edge_attr_chromophore, edge_attr_solvent
    del b1_c, b2_c, b1_s, b2_s, fc1_b

    src_c = edge_index_chromophore[0]
    src_s = edge_index_solvent[0]
    dst2_c = edge_index_chromophore[1].reshape(NCH, CH)
    dst2_s = edge_index_solvent[1].reshape(NCH, CH)
    ones16 = jnp.ones((CH, 16), jnp.float32)
    z16 = jnp.zeros((N, 16), jnp.float32)
    z128 = jnp.zeros((N, 128), jnp.float32)
    z64 = jnp.zeros((N, 64), jnp.float32)

    degp = _sc_degree(dst2_c, dst2_s, ones16, z16)

    h1 = _tc1(x_chromophore, x_solvent, jnp.stack([W1_c, W1_s]), degp)
    acc1 = _sc_conv128(h1, src_c, src_s, dst2_c, dst2_s, z128)
    h2 = _tc_layer1(acc1, h1, degp, jnp.stack([g1_c, g1_s]),
                    jnp.stack([be1_c, be1_s]), jnp.stack([W2_c, W2_s]))

    acc2 = _sc_conv64(h2, src_c, src_s, dst2_c, dst2_s, z64)
    pool, cnt = _tc_layer2(acc2, h2, degp, jnp.stack([g2_c, g2_s]),
                           jnp.stack([be2_c, be2_s]),
                           batch.reshape(NBLK, 1, BLK))

    fc2_Wp = jnp.pad(fc2_W, ((0, 0), (0, 127)))
    fc2_bp = jnp.pad(fc2_b, (0, 127))
    out = _tc4(pool, cnt, fc1_W, gfc, befc, fc2_Wp, fc2_bp)
    return out[:, :1]


# speedup vs baseline: 36.9458x; 1.0085x over previous
import functools

import jax
import jax.numpy as jnp
from jax import lax
from jax.experimental import pallas as pl
from jax.experimental.pallas import tpu as pltpu
from jax.experimental.pallas import tpu_sc as plsc

N = 10000
E = 320000
NG = 256
EPS = 1e-5

NC = 2
NS = 16
CH = 80
EPT = E // NS
NCHT = EPT // CH
NCH = E // CH
RPT = N // NS

NBLK = 1
BLK = N // NBLK


_MESH = plsc.VectorSubcoreMesh(core_axis_name="c", subcore_axis_name="s")
_SC_PARAMS = pltpu.CompilerParams(use_tc_tiling_on_sc=False)


@functools.partial(
    pl.kernel,
    out_type=jax.ShapeDtypeStruct((2 * N, 16), jnp.float32),
    mesh=_MESH,
    compiler_params=_SC_PARAMS,
    scratch_types=[
        pltpu.VMEM((NCHT, CH), jnp.int32),
        pltpu.VMEM((CH, 16), jnp.float32),
        pltpu.SemaphoreType.DMA,
        pltpu.VMEM_SHARED((N, 16), jnp.float32),
    ],
)
def _sc_degree(dstc_hbm, dsts_hbm, ones_hbm, zeros_hbm, out_hbm,
               dstb, onesb, sem, acc):
    cid = lax.axis_index("c")
    sid = lax.axis_index("s")

    @pl.when(cid == 0)
    def _():
        pltpu.sync_copy(dstc_hbm.at[pl.ds(sid * NCHT, NCHT)], dstb)

    @pl.when(cid == 1)
    def _():
        pltpu.sync_copy(dsts_hbm.at[pl.ds(sid * NCHT, NCHT)], dstb)

    pltpu.sync_copy(ones_hbm, onesb)
    r0 = sid * RPT
    pltpu.sync_copy(zeros_hbm.at[pl.ds(r0, RPT)], acc.at[pl.ds(r0, RPT)])
    plsc.subcore_barrier()

    def fire(j, carry):
        pltpu.async_copy(onesb, acc.at[dstb.at[j]], sem, add=True)
        return carry

    lax.fori_loop(0, NCHT, fire, 0)

    def drain(j, carry):
        pltpu.make_async_copy(onesb, acc.at[dstb.at[j]], sem).wait()
        return carry

    lax.fori_loop(0, NCHT, drain, 0)
    plsc.subcore_barrier()
    pltpu.sync_copy(acc.at[pl.ds(r0, RPT)], out_hbm.at[pl.ds(cid * N + r0, RPT)])


def _make_sc_conv(D):
    @functools.partial(
        pl.kernel,
        out_type=jax.ShapeDtypeStruct((2 * N, D), jnp.float32),
        mesh=_MESH,
        compiler_params=_SC_PARAMS,
        scratch_types=[
            pltpu.VMEM((EPT // 2,), jnp.int32),
            pltpu.VMEM((NCHT // 2, CH), jnp.int32),
            pltpu.VMEM((CH, D), jnp.float32),
            pltpu.VMEM((CH, D), jnp.float32),
            pltpu.VMEM((CH, D), jnp.float32),
            pltpu.SemaphoreType.DMA,
            pltpu.SemaphoreType.DMA,
            pltpu.SemaphoreType.DMA,
            pltpu.SemaphoreType.DMA,
            pltpu.SemaphoreType.DMA,
            pltpu.SemaphoreType.DMA,
            pltpu.VMEM_SHARED((N, D), jnp.float32),
        ],
    )
    def conv(h_hbm, srcc_hbm, srcs_hbm, dstc_hbm, dsts_hbm, zeros_hbm,
             out_hbm, srcb, dstb, b0, b1, b2, g0, g1, g2, s0, s1, s2, acc):
        cid = lax.axis_index("c")
        sid = lax.axis_index("s")
        r0 = sid * RPT
        pltpu.sync_copy(zeros_hbm.at[pl.ds(r0, RPT)], acc.at[pl.ds(r0, RPT)])
        plsc.subcore_barrier()
        bufs = (b0, b1, b2)
        gsems = (g0, g1, g2)
        ssems = (s0, s1, s2)
        table = h_hbm.at[pl.ds(cid * N, N)]

        def g_start(j, k):
            pltpu.async_copy(table.at[srcb.at[pl.ds(j * CH, CH)]],
                             bufs[k], gsems[k])

        def g_wait(j, k):
            pltpu.make_async_copy(table.at[srcb.at[pl.ds(j * CH, CH)]],
                                  bufs[k], gsems[k]).wait()

        def s_start(j, k):
            pltpu.async_copy(bufs[k], acc.at[dstb.at[j]], ssems[k], add=True)

        def s_drain(j, k):
            pltpu.make_async_copy(bufs[k], acc.at[dstb.at[j]],
                                  ssems[k]).wait()

        nh = NCHT // 2
        nloop = (nh - 2) // 3
        for h in range(2):
            @pl.when(cid == 0)
            def _(h=h):
                pltpu.sync_copy(
                    srcc_hbm.at[pl.ds(sid * EPT + h * (EPT // 2),
                                      EPT // 2)], srcb)
                pltpu.sync_copy(
                    dstc_hbm.at[pl.ds(sid * NCHT + h * nh, nh)], dstb)

            @pl.when(cid == 1)
            def _(h=h):
                pltpu.sync_copy(
                    srcs_hbm.at[pl.ds(sid * EPT + h * (EPT // 2),
                                      EPT // 2)], srcb)
                pltpu.sync_copy(
                    dsts_hbm.at[pl.ds(sid * NCHT + h * nh, nh)], dstb)
            for k in range(3):
                g_start(k, k)

            def step(t, carry):
                for k in range(3):
                    j = 3 * t + k
                    g_wait(j, k)
                    s_start(j, k)

                    @pl.when(j + 3 < nh)
                    def _(j=j, k=k):
                        s_drain(j, k)
                        g_start(j + 3, k)
                return carry

            lax.fori_loop(0, nloop, step, 0)
            jt = nh - 2
            g_wait(jt, 0)
            s_start(jt, 0)
            g_wait(jt + 1, 1)
            s_start(jt + 1, 1)
            s_drain(jt - 1, 2)
            s_drain(jt, 0)
            s_drain(jt + 1, 1)
        plsc.subcore_barrier()
        pltpu.sync_copy(acc.at[pl.ds(r0, RPT)],
                        out_hbm.at[pl.ds(cid * N + r0, RPT)])

    return conv


_sc_conv128 = _make_sc_conv(128)
_sc_conv64 = _make_sc_conv(64)


NSTEP = 2 * NBLK


def _row_spec(d):
    return pl.BlockSpec((BLK, d), lambda i: (i, 0))


def _br_spec(shape):
    nd = len(shape)
    return pl.BlockSpec((1,) + shape, lambda i, _n=nd: (i // NBLK,) + (0,) * _n)


def _full_spec(shape):
    nd = len(shape)
    return pl.BlockSpec(shape, lambda i, _n=nd: (0,) * _n)


def _dinv(deg_blk):
    return lax.rsqrt(deg_blk[:, :1] + 1.0)


def _branch_sums(part_full):
    b = pl.program_id(0) // NBLK
    sel = (lax.broadcasted_iota(jnp.int32, (NSTEP, 1, 1), 0) // NBLK == b)
    return jnp.sum(jnp.where(sel, part_full, 0.0), axis=0)


def _sel_row(v2d):
    b = pl.program_id(0) // NBLK
    sel = (lax.broadcasted_iota(jnp.int32, (2, 1), 0) == b)
    return jnp.sum(jnp.where(sel, v2d, 0.0), axis=0)


def _tc1_body(xc, xs, w1, dg, h_o):
    b = pl.program_id(0) // NBLK
    x = jnp.where(b == 0, xc[...], xs[...])
    h_o[...] = jnp.dot(x, w1[...][0],
                       preferred_element_type=jnp.float32) * _dinv(dg[...])


def _tc1(x_c, x_s, W1, degp):
    return pl.pallas_call(
        _tc1_body,
        grid=(NSTEP,),
        in_specs=[pl.BlockSpec((BLK, 128), lambda i: (i % NBLK, 0)),
                  pl.BlockSpec((BLK, 128), lambda i: (i % NBLK, 0)),
                  _br_spec((128, 128)), _row_spec(16)],
        out_specs=_row_spec(128),
        out_shape=jax.ShapeDtypeStruct((2 * N, 128), jnp.float32),
    )(x_c, x_s, W1, degp)


def _bn_apply(y, sums, g, be):
    mean = sums[0] / N
    var = sums[1] / N - mean * mean
    return (y - mean[None, :]) * lax.rsqrt(var[None, :] + EPS) * g[None, :] \
        + be[None, :]


def _accum_stats(y, sums_ref):
    b = pl.program_id(1) // NBLK

    @pl.when((pl.program_id(0) == 0) & (pl.program_id(1) == 0))
    def _():
        sums_ref[...] = jnp.zeros_like(sums_ref)

    upd = jnp.stack([jnp.sum(y, 0), jnp.sum(y * y, 0)])
    sel = (lax.broadcasted_iota(jnp.int32, (2, 1, 1), 0) == b)
    sums_ref[...] = sums_ref[...] + jnp.where(sel, upd[None], 0.0)


def _sel_branch_stats(sums_ref):
    b = pl.program_id(1) // NBLK
    sel = (lax.broadcasted_iota(jnp.int32, (2, 1, 1), 0) == b)
    return jnp.sum(jnp.where(sel, sums_ref[...], 0.0), axis=0)


def _layer1_body(ac, hc, dg, g1, be1, w2, h_o, ysc, sums):
    p = pl.program_id(0)

    @pl.when(p == 0)
    def _():
        i = pl.program_id(1)
        y = _dinv(dg[...]) * (ac[...] + hc[...])
        ysc[pl.ds(i * BLK, BLK), :] = y
        _accum_stats(y, sums)

    @pl.when(p == 1)
    def _():
        i = pl.program_id(1)
        y = ysc[pl.ds(i * BLK, BLK), :]
        x = jnp.maximum(_bn_apply(y, _sel_branch_stats(sums),
                                  _sel_row(g1[...]), _sel_row(be1[...])), 0.0)
        h_o[...] = jnp.dot(x, w2[...][0],
                           preferred_element_type=jnp.float32) * _dinv(dg[...])


def _tc_layer1(acc, h, degp, g1, be1, W2):
    return pl.pallas_call(
        _layer1_body,
        grid=(2, NSTEP),
        in_specs=[
            pl.BlockSpec((BLK, 128), lambda p, i: (i * (1 - p), 0)),
            pl.BlockSpec((BLK, 128), lambda p, i: (i * (1 - p), 0)),
            pl.BlockSpec((BLK, 16), lambda p, i: (i, 0)),
            pl.BlockSpec((2, 128), lambda p, i: (0, 0)),
            pl.BlockSpec((2, 128), lambda p, i: (0, 0)),
            pl.BlockSpec((1, 128, 64), lambda p, i: (i // NBLK, 0, 0)),
        ],
        out_specs=pl.BlockSpec((BLK, 64), lambda p, i: (i * p, 0)),
        out_shape=jax.ShapeDtypeStruct((2 * N, 64), jnp.float32),
        scratch_shapes=[pltpu.VMEM((2 * N, 128), jnp.float32),
                        pltpu.VMEM((2, 2, 128), jnp.float32)],
    )(acc, h, degp, g1, be1, W2)


def _layer2_body(ac, hc, dg, g2, be2, batch, pool_o, cnt_o, ysc, sums):
    p = pl.program_id(0)

    @pl.when(p == 0)
    def _():
        i = pl.program_id(1)
        y = _dinv(dg[...]) * (ac[...] + hc[...])
        ysc[pl.ds(i * BLK, BLK), :] = y
        _accum_stats(y, sums)

    @pl.when(p == 1)
    def _():
        i = pl.program_id(1)
        y = ysc[pl.ds(i * BLK, BLK), :]
        z = jnp.maximum(_bn_apply(y, _sel_branch_stats(sums),
                                  _sel_row(g2[...]), _sel_row(be2[...])), 0.0)
        gids = lax.broadcasted_iota(jnp.int32, (NG, BLK), 0)
        onehot = (batch[...][0] == gids).astype(jnp.float32)
        pool_o[...] = jnp.dot(onehot, z,
                              preferred_element_type=jnp.float32)[None]
        cnt_o[...] = jnp.broadcast_to(
            jnp.sum(onehot, axis=1, keepdims=True), (NG, 16))[None]


def _tc_layer2(acc, h, degp, g2, be2, batch3d):
    return pl.pallas_call(
        _layer2_body,
        grid=(2, NSTEP),
        in_specs=[
            pl.BlockSpec((BLK, 64), lambda p, i: (i * (1 - p), 0)),
            pl.BlockSpec((BLK, 64), lambda p, i: (i * (1 - p), 0)),
            pl.BlockSpec((BLK, 16), lambda p, i: (i, 0)),
            pl.BlockSpec((2, 64), lambda p, i: (0, 0)),
            pl.BlockSpec((2, 64), lambda p, i: (0, 0)),
            pl.BlockSpec((1, 1, BLK), lambda p, i: ((i % NBLK) * p, 0, 0)),
        ],
        out_specs=[pl.BlockSpec((1, NG, 64), lambda p, i: (i * p, 0, 0)),
                   pl.BlockSpec((1, NG, 16),
                                lambda p, i: ((i % NBLK) * p, 0, 0))],
        out_shape=[jax.ShapeDtypeStruct((NSTEP, NG, 64), jnp.float32),
                   jax.ShapeDtypeStruct((NBLK, NG, 16), jnp.float32)],
        scratch_shapes=[pltpu.VMEM((2 * N, 64), jnp.float32),
                        pltpu.VMEM((2, 2, 64), jnp.float32)],
    )(acc, h, degp, g2, be2, batch3d)


def _tc4_body(pool, cnt, fc1w, gfc, befc, fc2w, fc2b, out_o):
    pall = pool[...]
    c = jnp.maximum(jnp.sum(cnt[...], axis=0)[:, :1], 1.0)
    x = jnp.concatenate([jnp.sum(pall[:NBLK], 0) / c,
                         jnp.sum(pall[NBLK:], 0) / c], axis=1)
    h = jnp.dot(x, fc1w[...], preferred_element_type=jnp.float32)
    m = jnp.mean(h, axis=0)
    v = jnp.mean((h - m[None, :]) ** 2, axis=0)
    h = (h - m[None, :]) * lax.rsqrt(v[None, :] + EPS) * gfc[...][None, :] \
        + befc[...][None, :]
    h = jnp.maximum(h, 0.0)
    out_o[...] = jnp.dot(h, fc2w[...],
                         preferred_element_type=jnp.float32) + fc2b[...][None, :]


def _tc4(pool, cnt, fc1_W, gfc, befc, fc2_Wp, fc2_bp):
    return pl.pallas_call(
        _tc4_body,
        grid=(1,),
        in_specs=[_full_spec((NSTEP, NG, 64)), _full_spec((NBLK, NG, 16)),
                  _full_spec((128, 64)), _full_spec((64,)), _full_spec((64,)),
                  _full_spec((64, 128)), _full_spec((128,))],
        out_specs=_full_spec((NG, 128)),
        out_shape=jax.ShapeDtypeStruct((NG, 128), jnp.float32),
    )(pool, cnt, fc1_W, gfc, befc, fc2_Wp, fc2_bp)


def kernel(x_chromophore, edge_index_chromophore, edge_attr_chromophore,
           x_solvent, edge_index_solvent, edge_attr_solvent, batch,
           W1_c, b1_c, g1_c, be1_c, W2_c, b2_c, g2_c, be2_c,
           W1_s, b1_s, g1_s, be1_s, W2_s, b2_s, g2_s, be2_s,
           fc1_W, fc1_b, gfc, befc, fc2_W, fc2_b):
    del edge_attr_chromophore, edge_attr_solvent
    del b1_c, b2_c, b1_s, b2_s, fc1_b

    src_c = edge_index_chromophore[0]
    src_s = edge_index_solvent[0]
    dst2_c = edge_index_chromophore[1].reshape(NCH, CH)
    dst2_s = edge_index_solvent[1].reshape(NCH, CH)
    ones16 = jnp.ones((CH, 16), jnp.float32)
    z16 = jnp.zeros((N, 16), jnp.float32)
    z128 = jnp.zeros((N, 128), jnp.float32)
    z64 = jnp.zeros((N, 64), jnp.float32)

    degp = _sc_degree(dst2_c, dst2_s, ones16, z16)

    h1 = _tc1(x_chromophore, x_solvent, jnp.stack([W1_c, W1_s]), degp)
    acc1 = _sc_conv128(h1, src_c, src_s, dst2_c, dst2_s, z128)
    h2 = _tc_layer1(acc1, h1, degp, jnp.stack([g1_c, g1_s]),
                    jnp.stack([be1_c, be1_s]), jnp.stack([W2_c, W2_s]))

    acc2 = _sc_conv64(h2, src_c, src_s, dst2_c, dst2_s, z64)
    pool, cnt = _tc_layer2(acc2, h2, degp, jnp.stack([g2_c, g2_s]),
                           jnp.stack([be2_c, be2_s]),
                           batch.reshape(NBLK, 1, BLK))

    fc2_Wp = jnp.pad(fc2_W, ((0, 0), (0, 127)))
    fc2_bp = jnp.pad(fc2_b, (0, 127))
    out = _tc4(pool, cnt, fc1_W, gfc, befc, fc2_Wp, fc2_bp)
    return out[:, :1]
